# baseline (speedup 1.0000x reference)
import functools

import jax
import jax.numpy as jnp
from jax import lax
from jax.experimental import pallas as pl
from jax.experimental.pallas import tpu as pltpu
from jax.experimental.pallas import tpu_sc as plsc

N = 50000
E = 800000
D = 32
DE = 16

NC = 2
NS = 16
NW = NC * NS

CB = 128
EPW = 25600
E_PAD = NW * EPW
ER = E_PAD // 128
WR = EPW // 128
NPAIR = WR // 2

CBC = 1280
SUBC = CBC // 128
WRC = EPW // CBC
NPAIRC = WRC // 2

ROWS_PER_TILE = 3128
ACC_ROWS = NS * ROWS_PER_TILE

BE = 2048
BN = 2000

_mesh = plsc.VectorSubcoreMesh(core_axis_name="c", subcore_axis_name="s")
_sc_params = pltpu.CompilerParams(use_tc_tiling_on_sc=False)

_INV_2PI = 0.15915494309189535
_COS_C = (1.0, -19.739208221435547, 64.93938446044922, -85.45662689208984,
          60.24174118041992, -26.402328491210938, 7.793178081512451,
          -1.4450093507766724)


def _fast_cos(x):
    r = x * _INV_2PI
    r = r - jnp.round(r)
    u = r * r
    p = jnp.full_like(u, _COS_C[-1])
    for c in _COS_C[-2::-1]:
        p = p * u + c
    return p


QROWS = E_PAD // 4 // 128


def _tfw_body(r0_ref, r1_ref, r2_ref, r3_ref, w_ref, b_ref, wmtT_ref,
              cvec_ref, out_ref):
    parts = []
    for rel_ref in (r0_ref, r1_ref, r2_ref, r3_ref):
        rel = rel_ref[...].reshape(1, BE)
        tf = _fast_cos(w_ref[...] * rel + b_ref[...])
        t = jnp.dot(wmtT_ref[...], tf, preferred_element_type=jnp.float32)
        parts.append(t.T + cvec_ref[...])
    out_ref[...] = jnp.concatenate(parts, axis=1)


def _tfw_prep(rel_r, w_col, b_col, wmtT, cvec):
    g = E_PAD // 4 // BE
    full = lambda a: pl.BlockSpec(a.shape, lambda i: (0,) * a.ndim)
    rb = BE // 128
    rspecs = [pl.BlockSpec((rb, 128), functools.partial(
        lambda a, i: (i + a * (QROWS // rb), 0), a)) for a in range(4)]
    return pl.pallas_call(
        _tfw_body,
        grid=(g,),
        in_specs=rspecs + [full(w_col), full(b_col), full(wmtT), full(cvec)],
        out_specs=pl.BlockSpec((BE, 128), lambda i: (i, 0)),
        out_shape=jax.ShapeDtypeStruct((E_PAD // 4, 128), jnp.float32),
    )(rel_r, rel_r, rel_r, rel_r, w_col, b_col, wmtT, cvec)


def _dmask_body(mask_ref, dst_ref, d1_ref, d2_ref):
    m = mask_ref[...]
    d = dst_ref[...]
    d1_ref[...] = jnp.where(m != 0, d, N)
    d2_ref[...] = jnp.where(m != 2, d, N)


def _dmask(mask_r, dst_r):
    g = ER // 128
    return pl.pallas_call(
        _dmask_body,
        grid=(g,),
        in_specs=[pl.BlockSpec((128, 128), lambda i: (i, 0)),
                  pl.BlockSpec((128, 128), lambda i: (i, 0))],
        out_specs=[pl.BlockSpec((128, 128), lambda i: (i, 0)),
                   pl.BlockSpec((128, 128), lambda i: (i, 0))],
        out_shape=[jax.ShapeDtypeStruct((ER, 128), jnp.int32),
                   jax.ShapeDtypeStruct((ER, 128), jnp.int32)],
    )(mask_r, dst_r)


def _tem_body(e0_ref, e1_ref, e2_ref, e3_ref, wemT_ref, out_ref):
    parts = []
    for ef_ref in (e0_ref, e1_ref, e2_ref, e3_ref):
        t = jnp.dot(wemT_ref[...], ef_ref[...],
                    preferred_element_type=jnp.float32)
        parts.append(t.T)
    out_ref[...] = jnp.concatenate(parts, axis=1)


def _tem_prep(efT, wemT):
    g = E_PAD // 4 // BE
    full = lambda a: pl.BlockSpec(a.shape, lambda i: (0,) * a.ndim)
    especs = [pl.BlockSpec((DE, BE), functools.partial(
        lambda a, i: (0, i + a * g), a)) for a in range(4)]
    return pl.pallas_call(
        _tem_body,
        grid=(g,),
        in_specs=especs + [full(wemT)],
        out_specs=pl.BlockSpec((BE, 128), lambda i: (i, 0)),
        out_shape=jax.ShapeDtypeStruct((E_PAD // 4, 128), jnp.float32),
    )(efT, efT, efT, efT, wemT)


def _node_prep_body(mem_ref, ts_ref, w_ref, b_ref, wmh_ref, h0_ref, hw_ref):
    ts = ts_ref[...].reshape(1, BN)
    tf = _fast_cos(w_ref[...] * ts + b_ref[...])
    h0 = mem_ref[...] + tf.T
    h0_ref[...] = h0
    hw_ref[...] = jnp.dot(h0, wmh_ref[...], preferred_element_type=jnp.float32)


def _node_prep(memory, ts_row, w_col, b_col, wmh):
    g = N // BN
    full = lambda a: pl.BlockSpec(a.shape, lambda i: (0,) * a.ndim)
    return pl.pallas_call(
        _node_prep_body,
        grid=(g,),
        in_specs=[pl.BlockSpec((BN, D), lambda i: (i, 0)),
                  pl.BlockSpec((1, 1, BN), lambda i: (i, 0, 0)),
                  full(w_col), full(b_col), full(wmh)],
        out_specs=[pl.BlockSpec((BN, D), lambda i: (i, 0)),
                   pl.BlockSpec((BN, D), lambda i: (i, 0))],
        out_shape=[jax.ShapeDtypeStruct((N, D), jnp.float32),
                   jax.ShapeDtypeStruct((N, D), jnp.float32)],
    )(memory, ts_row, w_col, b_col, wmh)


def _update_body(h_ref, p0_ref, p1_ref, c0_ref, c1_ref, wlin_ref, wmh_ref,
                 h_out, hw_out):
    cnt = c0_ref[...][:, 0:1] + c1_ref[...][:, 0:1]
    agg = (p0_ref[...] + p1_ref[...]) / (cnt + 1.0)
    h = jax.nn.relu(jnp.dot(h_ref[...] + agg, wlin_ref[...],
                            preferred_element_type=jnp.float32))
    h_out[...] = h
    hw_out[...] = jnp.dot(h, wmh_ref[...], preferred_element_type=jnp.float32)


def _update(h, p0, p1, c0, c1, wlin, wmh):
    g = N // BN
    full = lambda a: pl.BlockSpec(a.shape, lambda i: (0,) * a.ndim)
    return pl.pallas_call(
        _update_body,
        grid=(g,),
        in_specs=[pl.BlockSpec((BN, D), lambda i: (i, 0)),
                  pl.BlockSpec((BN, D), lambda i: (i, 0)),
                  pl.BlockSpec((BN, D), lambda i: (i, 0)),
                  pl.BlockSpec((BN, 8), lambda i: (i, 0)),
                  pl.BlockSpec((BN, 8), lambda i: (i, 0)),
                  full(wlin), full(wmh)],
        out_specs=[pl.BlockSpec((BN, D), lambda i: (i, 0)),
                   pl.BlockSpec((BN, D), lambda i: (i, 0))],
        out_shape=[jax.ShapeDtypeStruct((N, D), jnp.float32),
                   jax.ShapeDtypeStruct((N, D), jnp.float32)],
    )(h, p0, p1, c0, c1, wlin, wmh)


def _final_body(h_ref, p0_ref, p1_ref, c0_ref, c1_ref, wlin_ref, w1_ref,
                b1_ref, w2_ref, b2_ref, w3_ref, b3_ref, out_ref):
    cnt = c0_ref[...][:, 0:1] + c1_ref[...][:, 0:1]
    agg = (p0_ref[...] + p1_ref[...]) / (cnt + 1.0)
    h = jax.nn.relu(jnp.dot(h_ref[...] + agg, wlin_ref[...],
                            preferred_element_type=jnp.float32))
    x = jax.nn.relu(jnp.dot(h, w1_ref[...],
                            preferred_element_type=jnp.float32) + b1_ref[...])
    x = jax.nn.relu(jnp.dot(x, w2_ref[...],
                            preferred_element_type=jnp.float32) + b2_ref[...])
    out_ref[...] = jnp.dot(x, w3_ref[...],
                           preferred_element_type=jnp.float32) + b3_ref[...]


def _final(h, p0, p1, c0, c1, wlin, w1, b1, w2, b2, w3, b3):
    g = N // BN
    full = lambda a: pl.BlockSpec(a.shape, lambda i: (0,) * a.ndim)
    return pl.pallas_call(
        _final_body,
        grid=(g,),
        in_specs=[pl.BlockSpec((BN, D), lambda i: (i, 0)),
                  pl.BlockSpec((BN, D), lambda i: (i, 0)),
                  pl.BlockSpec((BN, D), lambda i: (i, 0)),
                  pl.BlockSpec((BN, 8), lambda i: (i, 0)),
                  pl.BlockSpec((BN, 8), lambda i: (i, 0)),
                  full(wlin), full(w1), full(b1), full(w2), full(b2),
                  full(w3), full(b3)],
        out_specs=pl.BlockSpec((BN, 2), lambda i: (i, 0)),
        out_shape=jax.ShapeDtypeStruct((N, 2), jnp.float32),
    )(h, p0, p1, c0, c1, wlin, w1, b1, w2, b2, w3, b3)


RT = 128
NRT = (E_PAD // 4 // NW) // RT
NPRT = NRT // 2


def _counts_body(d1_hbm, d2_hbm, tem4_hbm, ones_hbm, z8_hbm,
                 c1_hbm, c2_hbm, tem_hbm,
                 acc1, acc2, d1a, d2a, d1b, d2b, ones_v, ra, rb,
                 ia, ib, sa, sb, rsa, rsb, wsa, wsb):
    cid = lax.axis_index("c")
    sid = lax.axis_index("s")
    wid = sid * NC + cid
    base = sid * ROWS_PER_TILE
    r0 = wid * WR

    pltpu.sync_copy(z8_hbm, acc1.at[pl.ds(base, ROWS_PER_TILE)])
    pltpu.sync_copy(z8_hbm, acc2.at[pl.ds(base, ROWS_PER_TILE)])
    pltpu.sync_copy(ones_hbm, ones_v)
    plsc.subcore_barrier()

    def lin_issue(bufs, k):
        d1v, d2v, isem, _ = bufs
        r = r0 + jnp.minimum(k, WRC - 1) * SUBC
        pltpu.async_copy(d1_hbm.at[pl.ds(r, SUBC)], d1v, isem)
        pltpu.async_copy(d2_hbm.at[pl.ds(r, SUBC)], d2v, isem)

    def lin_wait(bufs):
        d1v, d2v, isem, _ = bufs
        pltpu.make_async_copy(d1_hbm.at[pl.ds(r0, SUBC)], d1v, isem).wait()
        pltpu.make_async_copy(d2_hbm.at[pl.ds(r0, SUBC)], d2v, isem).wait()

    def scat_issue(bufs):
        d1v, d2v, _, ssem = bufs
        for j in range(SUBC):
            pltpu.async_copy(ones_v, acc1.at[d1v.at[j]], ssem, add=True)
            pltpu.async_copy(ones_v, acc2.at[d2v.at[j]], ssem, add=True)

    def scat_wait(bufs):
        d1v, d2v, _, ssem = bufs
        for j in range(SUBC):
            pltpu.make_async_copy(ones_v, acc1.at[d1v.at[j]], ssem).wait()
            pltpu.make_async_copy(ones_v, acc2.at[d2v.at[j]], ssem).wait()

    A = (d1a, d2a, ia, sa)
    B = (d1b, d2b, ib, sb)
    lin_issue(A, 0)
    lin_issue(B, 1)

    def body(i, carry):
        a = 2 * i
        lin_wait(A)
        scat_issue(A)
        lin_wait(B)
        scat_issue(B)
        scat_wait(A)
        lin_issue(A, a + 2)
        scat_wait(B)
        lin_issue(B, a + 3)
        return carry

    lax.fori_loop(0, NPAIRC, body, 0)
    lin_wait(A)
    lin_wait(B)

    rbase = wid * (E_PAD // 4 // NW)

    def rd_issue(buf, rsem, k):
        kk = jnp.minimum(k, NRT - 1)
        pltpu.async_copy(tem4_hbm.at[pl.ds(rbase + kk * RT, RT)], buf, rsem)

    def rd_wait(buf, rsem):
        pltpu.make_async_copy(tem4_hbm.at[pl.ds(rbase, RT)], buf,
                              rsem).wait()

    def wr_issue(buf, wsem, k):
        for a in range(4):
            pltpu.async_copy(
                buf.at[:, pl.ds(a * D, D)],
                tem_hbm.at[pl.ds(a * (E_PAD // 4) + rbase + k * RT, RT)],
                wsem)

    def wr_wait(buf, wsem):
        for a in range(4):
            pltpu.make_async_copy(
                buf.at[:, pl.ds(a * D, D)],
                tem_hbm.at[pl.ds(a * (E_PAD // 4) + rbase, RT)],
                wsem).wait()

    rd_issue(ra, rsa, 0)
    rd_issue(rb, rsb, 1)

    def dbody(i, carry):
        k = 2 * i
        rd_wait(ra, rsa)
        wr_issue(ra, wsa, k)
        rd_wait(rb, rsb)
        wr_issue(rb, wsb, k + 1)
        wr_wait(ra, wsa)
        rd_issue(ra, rsa, k + 2)
        wr_wait(rb, wsb)
        rd_issue(rb, rsb, k + 3)
        return carry

    lax.fori_loop(0, NPRT, dbody, 0)
    rd_wait(ra, rsa)
    rd_wait(rb, rsb)
    plsc.subcore_barrier()
    pltpu.sync_copy(acc1.at[pl.ds(base, ROWS_PER_TILE)],
                    c1_hbm.at[cid, pl.ds(base, ROWS_PER_TILE)])
    pltpu.sync_copy(acc2.at[pl.ds(base, ROWS_PER_TILE)],
                    c2_hbm.at[cid, pl.ds(base, ROWS_PER_TILE)])


def _counts(d1_r, d2_r, tem4, ones8, z8):
    f = pl.kernel(
        _counts_body,
        out_type=[jax.ShapeDtypeStruct((NC, ACC_ROWS, 8), jnp.float32),
                  jax.ShapeDtypeStruct((NC, ACC_ROWS, 8), jnp.float32),
                  jax.ShapeDtypeStruct((E_PAD, D), jnp.float32)],
        mesh=_mesh,
        scratch_types=[
            pltpu.VMEM_SHARED((ACC_ROWS, 8), jnp.float32),
            pltpu.VMEM_SHARED((ACC_ROWS, 8), jnp.float32),
            pltpu.VMEM((SUBC, 128), jnp.int32),
            pltpu.VMEM((SUBC, 128), jnp.int32),
            pltpu.VMEM((SUBC, 128), jnp.int32),
            pltpu.VMEM((SUBC, 128), jnp.int32),
            pltpu.VMEM((128, 8), jnp.float32),
            pltpu.VMEM((RT, 128), jnp.float32),
            pltpu.VMEM((RT, 128), jnp.float32),
            pltpu.SemaphoreType.DMA,
            pltpu.SemaphoreType.DMA,
            pltpu.SemaphoreType.DMA,
            pltpu.SemaphoreType.DMA,
            pltpu.SemaphoreType.DMA,
            pltpu.SemaphoreType.DMA,
            pltpu.SemaphoreType.DMA,
            pltpu.SemaphoreType.DMA,
        ],
        compiler_params=_sc_params,
    )
    return f(d1_r, d2_r, tem4, ones8, z8)


def _edge_pass_body(hw_hbm, tem_hbm, tfw_hbm, src_hbm, eidx_hbm, dst_hbm,
                    z32_hbm, p_hbm, acc,
                    sa, ea, da, tfa, ha, ta, sb, eb, db, tfb, hb, tb,
                    isa, isb, fsa, fsb, gsa, gsb, ssa, ssb):
    cid = lax.axis_index("c")
    sid = lax.axis_index("s")
    wid = sid * NC + cid
    base = sid * ROWS_PER_TILE
    r0 = wid * WR

    pltpu.sync_copy(z32_hbm, acc.at[pl.ds(base, ROWS_PER_TILE)])
    plsc.subcore_barrier()

    def se_issue(bufs, k):
        s, e, d, tf, h, t, isem, fsem, gsem, ssem = bufs
        r = r0 + jnp.minimum(k, WR - 1)
        pltpu.async_copy(src_hbm.at[pl.ds(r, 1)], s, isem)
        pltpu.async_copy(eidx_hbm.at[pl.ds(r, 1)], e, isem)

    def se_wait(bufs):
        s, e, d, tf, h, t, isem, fsem, gsem, ssem = bufs
        pltpu.make_async_copy(src_hbm.at[pl.ds(r0, 1)], s, isem).wait()
        pltpu.make_async_copy(eidx_hbm.at[pl.ds(r0, 1)], e, isem).wait()

    grp = wid // 8
    grow = (wid % 8) * EPW

    def dtf_issue(bufs, k):
        s, e, d, tf, h, t, isem, fsem, gsem, ssem = bufs
        kk = jnp.minimum(k, WR - 1)
        pltpu.async_copy(dst_hbm.at[pl.ds(r0 + kk, 1)], d, fsem)
        pltpu.async_copy(
            tfw_hbm.at[pl.ds(grow + kk * 128, CB), pl.ds(grp * D, D)],
            tf, fsem)

    def dtf_wait(bufs):
        s, e, d, tf, h, t, isem, fsem, gsem, ssem = bufs
        pltpu.make_async_copy(dst_hbm.at[pl.ds(r0, 1)], d, fsem).wait()
        pltpu.make_async_copy(
            tfw_hbm.at[pl.ds(grow, CB), pl.ds(grp * D, D)], tf, fsem).wait()

    def gat_issue(bufs):
        s, e, d, tf, h, t, isem, fsem, gsem, ssem = bufs
        pltpu.async_copy(hw_hbm.at[s.at[0]], h, gsem)
        pltpu.async_copy(tem_hbm.at[e.at[0]], t, gsem)

    def gat_wait(bufs):
        s, e, d, tf, h, t, isem, fsem, gsem, ssem = bufs
        pltpu.make_async_copy(hw_hbm.at[s.at[0]], h, gsem).wait()
        pltpu.make_async_copy(tem_hbm.at[e.at[0]], t, gsem).wait()

    def scat_issue(bufs):
        s, e, d, tf, h, t, isem, fsem, gsem, ssem = bufs
        pltpu.async_copy(tf, acc.at[d.at[0]], ssem, add=True)

    def scat_wait(bufs):
        s, e, d, tf, h, t, isem, fsem, gsem, ssem = bufs
        pltpu.make_async_copy(tf, acc.at[d.at[0]], ssem).wait()

    def compute(bufs):
        s, e, d, tf, h, t, isem, fsem, gsem, ssem = bufs

        @plsc.parallel_loop(0, CB, 1, unroll=8)
        def _(r):
            for half in (0, 16):
                v = (h[r, pl.ds(half, 16)] + t[r, pl.ds(half, 16)]
                     + tf[r, pl.ds(half, 16)])
                tf[r, pl.ds(half, 16)] = jnp.maximum(v, 0.0)

    A = (sa, ea, da, tfa, ha, ta, isa, fsa, gsa, ssa)
    B = (sb, eb, db, tfb, hb, tb, isb, fsb, gsb, ssb)

    se_issue(A, 0)
    dtf_issue(A, 0)
    se_issue(B, 1)
    se_wait(A)
    gat_issue(A)

    def body(i, carry):
        a = 2 * i

        se_wait(B)
        dtf_issue(B, a + 1)
        gat_issue(B)

        gat_wait(A)
        dtf_wait(A)
        compute(A)
        scat_issue(A)
        se_issue(A, a + 2)

        gat_wait(B)
        dtf_wait(B)
        compute(B)
        scat_issue(B)
        se_issue(B, a + 3)

        scat_wait(A)
        dtf_issue(A, a + 2)
        se_wait(A)
        gat_issue(A)

        scat_wait(B)
        return carry

    lax.fori_loop(0, NPAIR, body, 0)

    gat_wait(A)
    dtf_wait(A)
    se_wait(B)
    plsc.subcore_barrier()
    pltpu.sync_copy(acc.at[pl.ds(base, ROWS_PER_TILE)],
                    p_hbm.at[cid, pl.ds(base, ROWS_PER_TILE)])


def _edge_pass(hw, tem, tfw, src_r, eidx_r, dst_r, z32):
    f = pl.kernel(
        _edge_pass_body,
        out_type=jax.ShapeDtypeStruct((NC, ACC_ROWS, D), jnp.float32),
        mesh=_mesh,
        scratch_types=[
            pltpu.VMEM_SHARED((ACC_ROWS, D), jnp.float32),
            pltpu.VMEM((1, 128), jnp.int32),
            pltpu.VMEM((1, 128), jnp.int32),
            pltpu.VMEM((1, 128), jnp.int32),
            pltpu.VMEM((CB, D), jnp.float32),
            pltpu.VMEM((CB, D), jnp.float32),
            pltpu.VMEM((CB, D), jnp.float32),
            pltpu.VMEM((1, 128), jnp.int32),
            pltpu.VMEM((1, 128), jnp.int32),
            pltpu.VMEM((1, 128), jnp.int32),
            pltpu.VMEM((CB, D), jnp.float32),
            pltpu.VMEM((CB, D), jnp.float32),
            pltpu.VMEM((CB, D), jnp.float32),
            pltpu.SemaphoreType.DMA,
            pltpu.SemaphoreType.DMA,
            pltpu.SemaphoreType.DMA,
            pltpu.SemaphoreType.DMA,
            pltpu.SemaphoreType.DMA,
            pltpu.SemaphoreType.DMA,
            pltpu.SemaphoreType.DMA,
            pltpu.SemaphoreType.DMA,
        ],
        compiler_params=_sc_params,
    )
    return f(hw, tem, tfw, src_r, eidx_r, dst_r, z32)


def kernel(node_x, node_timestamp, edge_index, edge_rel_times, edge_idx,
           edge_mask, memory, edge_feature, w_t, b_t, W_e, b_e, W_msg, b_msg,
           W_lin, W1, b1, W2, b2, W3, b3):
    wmh = W_msg[0:D]
    wmt = W_msg[D:2 * D]
    wme = W_msg[2 * D:]
    wem = W_e @ wme
    cvec = (b_e @ wme + b_msg)[None, :]
    w_col = w_t.T
    b_col = b_t[:, None]

    src = edge_index[0]
    dst = edge_index[1]
    pe = E_PAD - E
    rel_r = jnp.pad(edge_rel_times, (0, pe)).reshape(ER, 128)
    mask_r = jnp.pad(edge_mask, (0, pe)).reshape(ER, 128)
    dstp_r = jnp.pad(dst, (0, pe), constant_values=N).reshape(ER, 128)
    src_r = jnp.pad(src, (0, pe)).reshape(ER, 128)
    eidx_r = jnp.pad(edge_idx, (0, pe)).reshape(ER, 128)

    z32 = jnp.zeros((ROWS_PER_TILE, D), jnp.float32)
    z8 = jnp.zeros((ROWS_PER_TILE, 8), jnp.float32)
    ones8 = jnp.zeros((128, 8), jnp.float32).at[:, 0].set(1.0)

    tfw = _tfw_prep(rel_r, w_col, b_col, wmt.T, cvec)
    d1_r, d2_r = _dmask(mask_r, dstp_r)
    efT = jnp.pad(edge_feature, ((0, pe), (0, 0))).T
    tem4 = _tem_prep(efT, wem.T)

    c1p, c2p, tem = _counts(d1_r, d2_r, tem4, ones8, z8)
    h0, hw0 = _node_prep(memory, node_timestamp.reshape(N // BN, 1, BN),
                         w_col, b_col, wmh)

    p1 = _edge_pass(hw0, tem, tfw, src_r, eidx_r, d1_r, z32)
    h1, hw1 = _update(h0, p1[0, :N], p1[1, :N], c1p[0, :N], c1p[1, :N],
                      W_lin, wmh)

    p2 = _edge_pass(hw1, tem, tfw, src_r, eidx_r, d2_r, z32)
    return _final(h1, p2[0, :N], p2[1, :N], c2p[0, :N], c2p[1, :N], W_lin,
                  W1, b1[None, :], W2, b2[None, :], W3, b3[None, :])


# speedup vs baseline: 1.0012x; 1.0012x over previous
import functools

import jax
import jax.numpy as jnp
from jax import lax
from jax.experimental import pallas as pl
from jax.experimental.pallas import tpu as pltpu
from jax.experimental.pallas import tpu_sc as plsc

N = 50000
E = 800000
D = 32
DE = 16

NC = 2
NS = 16
NW = NC * NS

CB = 128
EPW = 25600
E_PAD = NW * EPW
ER = E_PAD // 128
WR = EPW // 128
NPAIR = WR // 2

CBC = 1280
SUBC = CBC // 128
WRC = EPW // CBC
NPAIRC = WRC // 2

ROWS_PER_TILE = 3128
ACC_ROWS = NS * ROWS_PER_TILE

BE = 2048
BN = 2000

_mesh = plsc.VectorSubcoreMesh(core_axis_name="c", subcore_axis_name="s")
_sc_params = pltpu.CompilerParams(use_tc_tiling_on_sc=False)

_INV_2PI = 0.15915494309189535
_COS_C = (1.0, -19.739208221435547, 64.93938446044922, -85.45662689208984,
          60.24174118041992, -26.402328491210938, 7.793178081512451,
          -1.4450093507766724)


def _fast_cos(x):
    r = x * _INV_2PI
    r = r - jnp.round(r)
    u = r * r
    p = jnp.full_like(u, _COS_C[-1])
    for c in _COS_C[-2::-1]:
        p = p * u + c
    return p


QROWS = E_PAD // 4 // 128


def _tfw_body(r0_ref, r1_ref, r2_ref, r3_ref, w_ref, b_ref, wmtT_ref,
              cvec_ref, out_ref):
    parts = []
    for rel_ref in (r0_ref, r1_ref, r2_ref, r3_ref):
        rel = rel_ref[...].reshape(1, BE)
        tf = _fast_cos(w_ref[...] * rel + b_ref[...])
        t = jnp.dot(wmtT_ref[...], tf, preferred_element_type=jnp.float32)
        parts.append(t.T + cvec_ref[...])
    out_ref[...] = jnp.concatenate(parts, axis=1)


def _tfw_prep(rel_r, w_col, b_col, wmtT, cvec):
    g = E_PAD // 4 // BE
    full = lambda a: pl.BlockSpec(a.shape, lambda i: (0,) * a.ndim)
    rb = BE // 128
    rspecs = [pl.BlockSpec((rb, 128), functools.partial(
        lambda a, i: (i + a * (QROWS // rb), 0), a)) for a in range(4)]
    return pl.pallas_call(
        _tfw_body,
        grid=(g,),
        in_specs=rspecs + [full(w_col), full(b_col), full(wmtT), full(cvec)],
        out_specs=pl.BlockSpec((BE, 128), lambda i: (i, 0)),
        out_shape=jax.ShapeDtypeStruct((E_PAD // 4, 128), jnp.float32),
    )(rel_r, rel_r, rel_r, rel_r, w_col, b_col, wmtT, cvec)


def _dmask_body(mask_ref, dst_ref, d1_ref, d2_ref):
    m = mask_ref[...]
    d = dst_ref[...]
    d1_ref[...] = jnp.where(m != 0, d, N)
    d2_ref[...] = jnp.where(m != 2, d, N)


def _dmask(mask_r, dst_r):
    g = ER // 128
    return pl.pallas_call(
        _dmask_body,
        grid=(g,),
        in_specs=[pl.BlockSpec((128, 128), lambda i: (i, 0)),
                  pl.BlockSpec((128, 128), lambda i: (i, 0))],
        out_specs=[pl.BlockSpec((128, 128), lambda i: (i, 0)),
                   pl.BlockSpec((128, 128), lambda i: (i, 0))],
        out_shape=[jax.ShapeDtypeStruct((ER, 128), jnp.int32),
                   jax.ShapeDtypeStruct((ER, 128), jnp.int32)],
    )(mask_r, dst_r)


def _tem_body(e0_ref, e1_ref, e2_ref, e3_ref, wemT_ref, out_ref):
    parts = []
    for ef_ref in (e0_ref, e1_ref, e2_ref, e3_ref):
        t = jnp.dot(wemT_ref[...], ef_ref[...],
                    preferred_element_type=jnp.float32)
        parts.append(t.T)
    out_ref[...] = jnp.concatenate(parts, axis=1)


def _tem_prep(efT, wemT):
    g = E_PAD // 4 // BE
    full = lambda a: pl.BlockSpec(a.shape, lambda i: (0,) * a.ndim)
    especs = [pl.BlockSpec((DE, BE), functools.partial(
        lambda a, i: (0, i + a * g), a)) for a in range(4)]
    return pl.pallas_call(
        _tem_body,
        grid=(g,),
        in_specs=especs + [full(wemT)],
        out_specs=pl.BlockSpec((BE, 128), lambda i: (i, 0)),
        out_shape=jax.ShapeDtypeStruct((E_PAD // 4, 128), jnp.float32),
    )(efT, efT, efT, efT, wemT)


def _node_prep_body(mem_ref, ts_ref, w_ref, b_ref, wmh_ref, h0_ref, hw_ref):
    ts = ts_ref[...].reshape(1, BN)
    tf = _fast_cos(w_ref[...] * ts + b_ref[...])
    h0 = mem_ref[...] + tf.T
    h0_ref[...] = h0
    hw_ref[...] = jnp.dot(h0, wmh_ref[...], preferred_element_type=jnp.float32)


def _node_prep(memory, ts_row, w_col, b_col, wmh):
    g = N // BN
    full = lambda a: pl.BlockSpec(a.shape, lambda i: (0,) * a.ndim)
    return pl.pallas_call(
        _node_prep_body,
        grid=(g,),
        in_specs=[pl.BlockSpec((BN, D), lambda i: (i, 0)),
                  pl.BlockSpec((1, 1, BN), lambda i: (i, 0, 0)),
                  full(w_col), full(b_col), full(wmh)],
        out_specs=[pl.BlockSpec((BN, D), lambda i: (i, 0)),
                   pl.BlockSpec((BN, D), lambda i: (i, 0))],
        out_shape=[jax.ShapeDtypeStruct((N, D), jnp.float32),
                   jax.ShapeDtypeStruct((N, D), jnp.float32)],
    )(memory, ts_row, w_col, b_col, wmh)


def _update_body(h_ref, p0_ref, p1_ref, c0_ref, c1_ref, wlin_ref, wmh_ref,
                 h_out, hw_out):
    cnt = c0_ref[...][:, 0:1] + c1_ref[...][:, 0:1]
    agg = (p0_ref[...] + p1_ref[...]) / (cnt + 1.0)
    h = jax.nn.relu(jnp.dot(h_ref[...] + agg, wlin_ref[...],
                            preferred_element_type=jnp.float32))
    h_out[...] = h
    hw_out[...] = jnp.dot(h, wmh_ref[...], preferred_element_type=jnp.float32)


def _update(h, p0, p1, c0, c1, wlin, wmh):
    g = N // BN
    full = lambda a: pl.BlockSpec(a.shape, lambda i: (0,) * a.ndim)
    return pl.pallas_call(
        _update_body,
        grid=(g,),
        in_specs=[pl.BlockSpec((BN, D), lambda i: (i, 0)),
                  pl.BlockSpec((BN, D), lambda i: (i, 0)),
                  pl.BlockSpec((BN, D), lambda i: (i, 0)),
                  pl.BlockSpec((BN, 8), lambda i: (i, 0)),
                  pl.BlockSpec((BN, 8), lambda i: (i, 0)),
                  full(wlin), full(wmh)],
        out_specs=[pl.BlockSpec((BN, D), lambda i: (i, 0)),
                   pl.BlockSpec((BN, D), lambda i: (i, 0))],
        out_shape=[jax.ShapeDtypeStruct((N, D), jnp.float32),
                   jax.ShapeDtypeStruct((N, D), jnp.float32)],
    )(h, p0, p1, c0, c1, wlin, wmh)


def _final_body(h_ref, p0_ref, p1_ref, c0_ref, c1_ref, wlin_ref, w1_ref,
                b1_ref, w2_ref, b2_ref, w3_ref, b3_ref, out_ref):
    cnt = c0_ref[...][:, 0:1] + c1_ref[...][:, 0:1]
    agg = (p0_ref[...] + p1_ref[...]) / (cnt + 1.0)
    h = jax.nn.relu(jnp.dot(h_ref[...] + agg, wlin_ref[...],
                            preferred_element_type=jnp.float32))
    x = jax.nn.relu(jnp.dot(h, w1_ref[...],
                            preferred_element_type=jnp.float32) + b1_ref[...])
    x = jax.nn.relu(jnp.dot(x, w2_ref[...],
                            preferred_element_type=jnp.float32) + b2_ref[...])
    out_ref[...] = jnp.dot(x, w3_ref[...],
                           preferred_element_type=jnp.float32) + b3_ref[...]


def _final(h, p0, p1, c0, c1, wlin, w1, b1, w2, b2, w3, b3):
    g = N // BN
    full = lambda a: pl.BlockSpec(a.shape, lambda i: (0,) * a.ndim)
    return pl.pallas_call(
        _final_body,
        grid=(g,),
        in_specs=[pl.BlockSpec((BN, D), lambda i: (i, 0)),
                  pl.BlockSpec((BN, D), lambda i: (i, 0)),
                  pl.BlockSpec((BN, D), lambda i: (i, 0)),
                  pl.BlockSpec((BN, 8), lambda i: (i, 0)),
                  pl.BlockSpec((BN, 8), lambda i: (i, 0)),
                  full(wlin), full(w1), full(b1), full(w2), full(b2),
                  full(w3), full(b3)],
        out_specs=pl.BlockSpec((BN, 2), lambda i: (i, 0)),
        out_shape=jax.ShapeDtypeStruct((N, 2), jnp.float32),
    )(h, p0, p1, c0, c1, wlin, w1, b1, w2, b2, w3, b3)


RT = 128
NRT = (E_PAD // 4 // NW) // RT
NPRT = NRT // 2


def _counts_body(d1_hbm, d2_hbm, tem4_hbm, ones_hbm, z8_hbm,
                 c1_hbm, c2_hbm, tem_hbm,
                 acc1, acc2, d1a, d2a, d1b, d2b, ones_v, ra, rb,
                 ia, ib, sa, sb, rsa, rsb, wsa, wsb):
    cid = lax.axis_index("c")
    sid = lax.axis_index("s")
    wid = sid * NC + cid
    base = sid * ROWS_PER_TILE
    r0 = wid * WR

    pltpu.sync_copy(z8_hbm, acc1.at[pl.ds(base, ROWS_PER_TILE)])
    pltpu.sync_copy(z8_hbm, acc2.at[pl.ds(base, ROWS_PER_TILE)])
    pltpu.sync_copy(ones_hbm, ones_v)
    plsc.subcore_barrier()

    def lin_issue(bufs, k):
        d1v, d2v, isem, _ = bufs
        r = r0 + jnp.minimum(k, WRC - 1) * SUBC
        pltpu.async_copy(d1_hbm.at[pl.ds(r, SUBC)], d1v, isem)
        pltpu.async_copy(d2_hbm.at[pl.ds(r, SUBC)], d2v, isem)

    def lin_wait(bufs):
        d1v, d2v, isem, _ = bufs
        pltpu.make_async_copy(d1_hbm.at[pl.ds(r0, SUBC)], d1v, isem).wait()
        pltpu.make_async_copy(d2_hbm.at[pl.ds(r0, SUBC)], d2v, isem).wait()

    def scat_issue(bufs):
        d1v, d2v, _, ssem = bufs
        for j in range(SUBC):
            pltpu.async_copy(ones_v, acc1.at[d1v.at[j]], ssem, add=True)
            pltpu.async_copy(ones_v, acc2.at[d2v.at[j]], ssem, add=True)

    def scat_wait(bufs):
        d1v, d2v, _, ssem = bufs
        for j in range(SUBC):
            pltpu.make_async_copy(ones_v, acc1.at[d1v.at[j]], ssem).wait()
            pltpu.make_async_copy(ones_v, acc2.at[d2v.at[j]], ssem).wait()

    A = (d1a, d2a, ia, sa)
    B = (d1b, d2b, ib, sb)
    lin_issue(A, 0)
    lin_issue(B, 1)

    def body(i, carry):
        a = 2 * i
        lin_wait(A)
        scat_issue(A)
        lin_wait(B)
        scat_issue(B)
        scat_wait(A)
        lin_issue(A, a + 2)
        scat_wait(B)
        lin_issue(B, a + 3)
        return carry

    lax.fori_loop(0, NPAIRC, body, 0)
    lin_wait(A)
    lin_wait(B)

    rbase = wid * (E_PAD // 4 // NW)

    def rd_issue(buf, rsem, k):
        kk = jnp.minimum(k, NRT - 1)
        pltpu.async_copy(tem4_hbm.at[pl.ds(rbase + kk * RT, RT)], buf, rsem)

    def rd_wait(buf, rsem):
        pltpu.make_async_copy(tem4_hbm.at[pl.ds(rbase, RT)], buf,
                              rsem).wait()

    def wr_issue(buf, wsem, k):
        for a in range(4):
            pltpu.async_copy(
                buf.at[:, pl.ds(a * D, D)],
                tem_hbm.at[pl.ds(a * (E_PAD // 4) + rbase + k * RT, RT)],
                wsem)

    def wr_wait(buf, wsem):
        for a in range(4):
            pltpu.make_async_copy(
                buf.at[:, pl.ds(a * D, D)],
                tem_hbm.at[pl.ds(a * (E_PAD // 4) + rbase, RT)],
                wsem).wait()

    rd_issue(ra, rsa, 0)
    rd_issue(rb, rsb, 1)

    def dbody(i, carry):
        k = 2 * i
        rd_wait(ra, rsa)
        wr_issue(ra, wsa, k)
        rd_wait(rb, rsb)
        wr_issue(rb, wsb, k + 1)
        wr_wait(ra, wsa)
        rd_issue(ra, rsa, k + 2)
        wr_wait(rb, wsb)
        rd_issue(rb, rsb, k + 3)
        return carry

    lax.fori_loop(0, NPRT, dbody, 0)
    rd_wait(ra, rsa)
    rd_wait(rb, rsb)
    plsc.subcore_barrier()
    pltpu.sync_copy(acc1.at[pl.ds(base, ROWS_PER_TILE)],
                    c1_hbm.at[cid, pl.ds(base, ROWS_PER_TILE)])
    pltpu.sync_copy(acc2.at[pl.ds(base, ROWS_PER_TILE)],
                    c2_hbm.at[cid, pl.ds(base, ROWS_PER_TILE)])


def _counts(d1_r, d2_r, tem4, ones8, z8):
    f = pl.kernel(
        _counts_body,
        out_type=[jax.ShapeDtypeStruct((NC, ACC_ROWS, 8), jnp.float32),
                  jax.ShapeDtypeStruct((NC, ACC_ROWS, 8), jnp.float32),
                  jax.ShapeDtypeStruct((E_PAD, D), jnp.float32)],
        mesh=_mesh,
        scratch_types=[
            pltpu.VMEM_SHARED((ACC_ROWS, 8), jnp.float32),
            pltpu.VMEM_SHARED((ACC_ROWS, 8), jnp.float32),
            pltpu.VMEM((SUBC, 128), jnp.int32),
            pltpu.VMEM((SUBC, 128), jnp.int32),
            pltpu.VMEM((SUBC, 128), jnp.int32),
            pltpu.VMEM((SUBC, 128), jnp.int32),
            pltpu.VMEM((128, 8), jnp.float32),
            pltpu.VMEM((RT, 128), jnp.float32),
            pltpu.VMEM((RT, 128), jnp.float32),
            pltpu.SemaphoreType.DMA,
            pltpu.SemaphoreType.DMA,
            pltpu.SemaphoreType.DMA,
            pltpu.SemaphoreType.DMA,
            pltpu.SemaphoreType.DMA,
            pltpu.SemaphoreType.DMA,
            pltpu.SemaphoreType.DMA,
            pltpu.SemaphoreType.DMA,
        ],
        compiler_params=_sc_params,
    )
    return f(d1_r, d2_r, tem4, ones8, z8)


def _edge_pass_body(hw_hbm, tem_hbm, tfw_hbm, src_hbm, eidx_hbm, dst_hbm,
                    z32_hbm, p_hbm, acc,
                    sa, ea, da, tfa, ha, ta, sb, eb, db, tfb, hb, tb,
                    isa, isb, fsa, fsb, gsa, gsb, ssa, ssb):
    cid = lax.axis_index("c")
    sid = lax.axis_index("s")
    wid = sid * NC + cid
    base = sid * ROWS_PER_TILE
    r0 = wid * WR

    pltpu.sync_copy(z32_hbm, acc.at[pl.ds(base, ROWS_PER_TILE)])
    plsc.subcore_barrier()

    def se_issue(bufs, k):
        s, e, d, tf, h, t, isem, fsem, gsem, ssem = bufs
        r = r0 + jnp.minimum(k, WR - 1)
        pltpu.async_copy(src_hbm.at[pl.ds(r, 1)], s, isem)
        pltpu.async_copy(eidx_hbm.at[pl.ds(r, 1)], e, isem)

    def se_wait(bufs):
        s, e, d, tf, h, t, isem, fsem, gsem, ssem = bufs
        pltpu.make_async_copy(src_hbm.at[pl.ds(r0, 1)], s, isem).wait()
        pltpu.make_async_copy(eidx_hbm.at[pl.ds(r0, 1)], e, isem).wait()

    grp = wid // 8
    grow = (wid % 8) * EPW

    def dtf_issue(bufs, k):
        s, e, d, tf, h, t, isem, fsem, gsem, ssem = bufs
        kk = jnp.minimum(k, WR - 1)
        pltpu.async_copy(dst_hbm.at[pl.ds(r0 + kk, 1)], d, fsem)
        pltpu.async_copy(
            tfw_hbm.at[pl.ds(grow + kk * 128, CB), pl.ds(grp * D, D)],
            tf, fsem)

    def dtf_wait(bufs):
        s, e, d, tf, h, t, isem, fsem, gsem, ssem = bufs
        pltpu.make_async_copy(dst_hbm.at[pl.ds(r0, 1)], d, fsem).wait()
        pltpu.make_async_copy(
            tfw_hbm.at[pl.ds(grow, CB), pl.ds(grp * D, D)], tf, fsem).wait()

    def gat_issue(bufs):
        s, e, d, tf, h, t, isem, fsem, gsem, ssem = bufs
        pltpu.async_copy(hw_hbm.at[s.at[0]], h, gsem)
        pltpu.async_copy(tem_hbm.at[e.at[0]], t, gsem)

    def gat_wait(bufs):
        s, e, d, tf, h, t, isem, fsem, gsem, ssem = bufs
        pltpu.make_async_copy(hw_hbm.at[s.at[0]], h, gsem).wait()
        pltpu.make_async_copy(tem_hbm.at[e.at[0]], t, gsem).wait()

    def scat_issue(bufs):
        s, e, d, tf, h, t, isem, fsem, gsem, ssem = bufs
        pltpu.async_copy(tf, acc.at[d.at[0]], ssem, add=True)

    def scat_wait(bufs):
        s, e, d, tf, h, t, isem, fsem, gsem, ssem = bufs
        pltpu.make_async_copy(tf, acc.at[d.at[0]], ssem).wait()

    def compute(bufs):
        s, e, d, tf, h, t, isem, fsem, gsem, ssem = bufs

        @plsc.parallel_loop(0, CB, 1, unroll=4)
        def _(r):
            for half in (0, 16):
                v = (h[r, pl.ds(half, 16)] + t[r, pl.ds(half, 16)]
                     + tf[r, pl.ds(half, 16)])
                tf[r, pl.ds(half, 16)] = jnp.maximum(v, 0.0)

    A = (sa, ea, da, tfa, ha, ta, isa, fsa, gsa, ssa)
    B = (sb, eb, db, tfb, hb, tb, isb, fsb, gsb, ssb)

    se_issue(A, 0)
    dtf_issue(A, 0)
    se_issue(B, 1)
    se_wait(A)
    gat_issue(A)

    def body(i, carry):
        a = 2 * i

        se_wait(B)
        dtf_issue(B, a + 1)
        gat_issue(B)

        gat_wait(A)
        dtf_wait(A)
        compute(A)
        scat_issue(A)
        se_issue(A, a + 2)

        gat_wait(B)
        dtf_wait(B)
        compute(B)
        scat_issue(B)
        se_issue(B, a + 3)

        scat_wait(A)
        dtf_issue(A, a + 2)
        se_wait(A)
        gat_issue(A)

        scat_wait(B)
        return carry

    lax.fori_loop(0, NPAIR, body, 0)

    gat_wait(A)
    dtf_wait(A)
    se_wait(B)
    plsc.subcore_barrier()
    pltpu.sync_copy(acc.at[pl.ds(base, ROWS_PER_TILE)],
                    p_hbm.at[cid, pl.ds(base, ROWS_PER_TILE)])


def _edge_pass(hw, tem, tfw, src_r, eidx_r, dst_r, z32):
    f = pl.kernel(
        _edge_pass_body,
        out_type=jax.ShapeDtypeStruct((NC, ACC_ROWS, D), jnp.float32),
        mesh=_mesh,
        scratch_types=[
            pltpu.VMEM_SHARED((ACC_ROWS, D), jnp.float32),
            pltpu.VMEM((1, 128), jnp.int32),
            pltpu.VMEM((1, 128), jnp.int32),
            pltpu.VMEM((1, 128), jnp.int32),
            pltpu.VMEM((CB, D), jnp.float32),
            pltpu.VMEM((CB, D), jnp.float32),
            pltpu.VMEM((CB, D), jnp.float32),
            pltpu.VMEM((1, 128), jnp.int32),
            pltpu.VMEM((1, 128), jnp.int32),
            pltpu.VMEM((1, 128), jnp.int32),
            pltpu.VMEM((CB, D), jnp.float32),
            pltpu.VMEM((CB, D), jnp.float32),
            pltpu.VMEM((CB, D), jnp.float32),
            pltpu.SemaphoreType.DMA,
            pltpu.SemaphoreType.DMA,
            pltpu.SemaphoreType.DMA,
            pltpu.SemaphoreType.DMA,
            pltpu.SemaphoreType.DMA,
            pltpu.SemaphoreType.DMA,
            pltpu.SemaphoreType.DMA,
            pltpu.SemaphoreType.DMA,
        ],
        compiler_params=_sc_params,
    )
    return f(hw, tem, tfw, src_r, eidx_r, dst_r, z32)


def kernel(node_x, node_timestamp, edge_index, edge_rel_times, edge_idx,
           edge_mask, memory, edge_feature, w_t, b_t, W_e, b_e, W_msg, b_msg,
           W_lin, W1, b1, W2, b2, W3, b3):
    wmh = W_msg[0:D]
    wmt = W_msg[D:2 * D]
    wme = W_msg[2 * D:]
    wem = W_e @ wme
    cvec = (b_e @ wme + b_msg)[None, :]
    w_col = w_t.T
    b_col = b_t[:, None]

    src = edge_index[0]
    dst = edge_index[1]
    pe = E_PAD - E
    rel_r = jnp.pad(edge_rel_times, (0, pe)).reshape(ER, 128)
    mask_r = jnp.pad(edge_mask, (0, pe)).reshape(ER, 128)
    dstp_r = jnp.pad(dst, (0, pe), constant_values=N).reshape(ER, 128)
    src_r = jnp.pad(src, (0, pe)).reshape(ER, 128)
    eidx_r = jnp.pad(edge_idx, (0, pe)).reshape(ER, 128)

    z32 = jnp.zeros((ROWS_PER_TILE, D), jnp.float32)
    z8 = jnp.zeros((ROWS_PER_TILE, 8), jnp.float32)
    ones8 = jnp.zeros((128, 8), jnp.float32).at[:, 0].set(1.0)

    tfw = _tfw_prep(rel_r, w_col, b_col, wmt.T, cvec)
    d1_r, d2_r = _dmask(mask_r, dstp_r)
    efT = jnp.pad(edge_feature, ((0, pe), (0, 0))).T
    tem4 = _tem_prep(efT, wem.T)

    c1p, c2p, tem = _counts(d1_r, d2_r, tem4, ones8, z8)
    h0, hw0 = _node_prep(memory, node_timestamp.reshape(N // BN, 1, BN),
                         w_col, b_col, wmh)

    p1 = _edge_pass(hw0, tem, tfw, src_r, eidx_r, d1_r, z32)
    h1, hw1 = _update(h0, p1[0, :N], p1[1, :N], c1p[0, :N], c1p[1, :N],
                      W_lin, wmh)

    p2 = _edge_pass(hw1, tem, tfw, src_r, eidx_r, d2_r, z32)
    return _final(h1, p2[0, :N], p2[1, :N], c2p[0, :N], c2p[1, :N], W_lin,
                  W1, b1[None, :], W2, b2[None, :], W3, b3[None, :])


# speedup vs baseline: 1.0471x; 1.0458x over previous
import functools

import jax
import jax.numpy as jnp
from jax import lax
from jax.experimental import pallas as pl
from jax.experimental.pallas import tpu as pltpu
from jax.experimental.pallas import tpu_sc as plsc

N = 50000
E = 800000
D = 32
DE = 16

NC = 2
NS = 16
NW = NC * NS

CB = 128
EPW = 25600
E_PAD = NW * EPW
ER = E_PAD // 128
WR = EPW // 128
NPAIR = WR // 2

CBC = 1280
SUBC = CBC // 128
WRC = EPW // CBC
NPAIRC = WRC // 2

ROWS_PER_TILE = 3128
ACC_ROWS = NS * ROWS_PER_TILE

BE = 2048
BN = 2000

_mesh = plsc.VectorSubcoreMesh(core_axis_name="c", subcore_axis_name="s")
_sc_params = pltpu.CompilerParams(use_tc_tiling_on_sc=False)

_INV_2PI = 0.15915494309189535
_COS_C = (1.0, -19.739208221435547, 64.93938446044922, -85.45662689208984,
          60.24174118041992, -26.402328491210938, 7.793178081512451,
          -1.4450093507766724)


def _fast_cos(x):
    r = x * _INV_2PI
    r = r - jnp.round(r)
    u = r * r
    p = jnp.full_like(u, _COS_C[-1])
    for c in _COS_C[-2::-1]:
        p = p * u + c
    return p


QROWS = E_PAD // 4 // 128


def _tfw_body(r0_ref, r1_ref, r2_ref, r3_ref, w_ref, b_ref, wmtT_ref,
              cvec_ref, out_ref):
    parts = []
    for rel_ref in (r0_ref, r1_ref, r2_ref, r3_ref):
        rel = rel_ref[...].reshape(1, BE)
        tf = _fast_cos(w_ref[...] * rel + b_ref[...])
        t = jnp.dot(wmtT_ref[...], tf, preferred_element_type=jnp.float32)
        parts.append(t.T + cvec_ref[...])
    out_ref[...] = jnp.concatenate(parts, axis=1)


def _tfw_prep(rel_r, w_col, b_col, wmtT, cvec):
    g = E_PAD // 4 // BE
    full = lambda a: pl.BlockSpec(a.shape, lambda i: (0,) * a.ndim)
    rb = BE // 128
    rspecs = [pl.BlockSpec((rb, 128), functools.partial(
        lambda a, i: (i + a * (QROWS // rb), 0), a)) for a in range(4)]
    return pl.pallas_call(
        _tfw_body,
        grid=(g,),
        in_specs=rspecs + [full(w_col), full(b_col), full(wmtT), full(cvec)],
        out_specs=pl.BlockSpec((BE, 128), lambda i: (i, 0)),
        out_shape=jax.ShapeDtypeStruct((E_PAD // 4, 128), jnp.float32),
    )(rel_r, rel_r, rel_r, rel_r, w_col, b_col, wmtT, cvec)


def _dmask_body(mask_ref, dst_ref, d1_ref, d2_ref):
    m = mask_ref[...]
    d = dst_ref[...]
    d1_ref[...] = jnp.where(m != 0, d, N)
    d2_ref[...] = jnp.where(m != 2, d, N)


def _dmask(mask_r, dst_r):
    g = ER // 128
    return pl.pallas_call(
        _dmask_body,
        grid=(g,),
        in_specs=[pl.BlockSpec((128, 128), lambda i: (i, 0)),
                  pl.BlockSpec((128, 128), lambda i: (i, 0))],
        out_specs=[pl.BlockSpec((128, 128), lambda i: (i, 0)),
                   pl.BlockSpec((128, 128), lambda i: (i, 0))],
        out_shape=[jax.ShapeDtypeStruct((ER, 128), jnp.int32),
                   jax.ShapeDtypeStruct((ER, 128), jnp.int32)],
    )(mask_r, dst_r)


def _tem_body(e0_ref, e1_ref, e2_ref, e3_ref, wemT_ref, out_ref):
    parts = []
    for ef_ref in (e0_ref, e1_ref, e2_ref, e3_ref):
        t = jnp.dot(wemT_ref[...], ef_ref[...],
                    preferred_element_type=jnp.float32)
        parts.append(t.T)
    out_ref[...] = jnp.concatenate(parts, axis=1)


def _tem_prep(efT, wemT):
    g = E_PAD // 4 // BE
    full = lambda a: pl.BlockSpec(a.shape, lambda i: (0,) * a.ndim)
    especs = [pl.BlockSpec((DE, BE), functools.partial(
        lambda a, i: (0, i + a * g), a)) for a in range(4)]
    return pl.pallas_call(
        _tem_body,
        grid=(g,),
        in_specs=especs + [full(wemT)],
        out_specs=pl.BlockSpec((BE, 128), lambda i: (i, 0)),
        out_shape=jax.ShapeDtypeStruct((E_PAD // 4, 128), jnp.float32),
    )(efT, efT, efT, efT, wemT)


def _node_prep_body(mem_ref, ts_ref, w_ref, b_ref, wmh_ref, h0_ref, hw_ref):
    ts = ts_ref[...].reshape(1, BN)
    tf = _fast_cos(w_ref[...] * ts + b_ref[...])
    h0 = mem_ref[...] + tf.T
    h0_ref[...] = h0
    hw_ref[...] = jnp.dot(h0, wmh_ref[...], preferred_element_type=jnp.float32)


def _node_prep(memory, ts_row, w_col, b_col, wmh):
    g = N // BN
    full = lambda a: pl.BlockSpec(a.shape, lambda i: (0,) * a.ndim)
    return pl.pallas_call(
        _node_prep_body,
        grid=(g,),
        in_specs=[pl.BlockSpec((BN, D), lambda i: (i, 0)),
                  pl.BlockSpec((1, 1, BN), lambda i: (i, 0, 0)),
                  full(w_col), full(b_col), full(wmh)],
        out_specs=[pl.BlockSpec((BN, D), lambda i: (i, 0)),
                   pl.BlockSpec((BN, D), lambda i: (i, 0))],
        out_shape=[jax.ShapeDtypeStruct((N, D), jnp.float32),
                   jax.ShapeDtypeStruct((N, D), jnp.float32)],
    )(memory, ts_row, w_col, b_col, wmh)


def _update_body(h_ref, p0_ref, p1_ref, c0_ref, c1_ref, wlin_ref, wmh_ref,
                 h_out, hw_out):
    cnt = c0_ref[...][:, 0:1] + c1_ref[...][:, 0:1]
    agg = (p0_ref[...] + p1_ref[...]) / (cnt + 1.0)
    h = jax.nn.relu(jnp.dot(h_ref[...] + agg, wlin_ref[...],
                            preferred_element_type=jnp.float32))
    h_out[...] = h
    hw_out[...] = jnp.dot(h, wmh_ref[...], preferred_element_type=jnp.float32)


def _update(h, p0, p1, c0, c1, wlin, wmh):
    g = N // BN
    full = lambda a: pl.BlockSpec(a.shape, lambda i: (0,) * a.ndim)
    return pl.pallas_call(
        _update_body,
        grid=(g,),
        in_specs=[pl.BlockSpec((BN, D), lambda i: (i, 0)),
                  pl.BlockSpec((BN, D), lambda i: (i, 0)),
                  pl.BlockSpec((BN, D), lambda i: (i, 0)),
                  pl.BlockSpec((BN, 8), lambda i: (i, 0)),
                  pl.BlockSpec((BN, 8), lambda i: (i, 0)),
                  full(wlin), full(wmh)],
        out_specs=[pl.BlockSpec((BN, D), lambda i: (i, 0)),
                   pl.BlockSpec((BN, D), lambda i: (i, 0))],
        out_shape=[jax.ShapeDtypeStruct((N, D), jnp.float32),
                   jax.ShapeDtypeStruct((N, D), jnp.float32)],
    )(h, p0, p1, c0, c1, wlin, wmh)


def _final_body(h_ref, p0_ref, p1_ref, c0_ref, c1_ref, wlin_ref, w1_ref,
                b1_ref, w2_ref, b2_ref, w3_ref, b3_ref, out_ref):
    cnt = c0_ref[...][:, 0:1] + c1_ref[...][:, 0:1]
    agg = (p0_ref[...] + p1_ref[...]) / (cnt + 1.0)
    h = jax.nn.relu(jnp.dot(h_ref[...] + agg, wlin_ref[...],
                            preferred_element_type=jnp.float32))
    x = jax.nn.relu(jnp.dot(h, w1_ref[...],
                            preferred_element_type=jnp.float32) + b1_ref[...])
    x = jax.nn.relu(jnp.dot(x, w2_ref[...],
                            preferred_element_type=jnp.float32) + b2_ref[...])
    out_ref[...] = jnp.dot(x, w3_ref[...],
                           preferred_element_type=jnp.float32) + b3_ref[...]


def _final(h, p0, p1, c0, c1, wlin, w1, b1, w2, b2, w3, b3):
    g = N // BN
    full = lambda a: pl.BlockSpec(a.shape, lambda i: (0,) * a.ndim)
    return pl.pallas_call(
        _final_body,
        grid=(g,),
        in_specs=[pl.BlockSpec((BN, D), lambda i: (i, 0)),
                  pl.BlockSpec((BN, D), lambda i: (i, 0)),
                  pl.BlockSpec((BN, D), lambda i: (i, 0)),
                  pl.BlockSpec((BN, 8), lambda i: (i, 0)),
                  pl.BlockSpec((BN, 8), lambda i: (i, 0)),
                  full(wlin), full(w1), full(b1), full(w2), full(b2),
                  full(w3), full(b3)],
        out_specs=pl.BlockSpec((BN, 2), lambda i: (i, 0)),
        out_shape=jax.ShapeDtypeStruct((N, 2), jnp.float32),
    )(h, p0, p1, c0, c1, wlin, w1, b1, w2, b2, w3, b3)


RT = 512
NRT = EPW // RT
NPRT = NRT // 2


def _counts_body(d1_hbm, d2_hbm, tem4_hbm, ones_hbm, z8_hbm,
                 c1_hbm, c2_hbm, tem_hbm,
                 acc1, acc2, d1a, d2a, d1b, d2b, ones_v, ra, rb,
                 ia, ib, sa, sb, rsa, rsb, wsa, wsb):
    cid = lax.axis_index("c")
    sid = lax.axis_index("s")
    wid = sid * NC + cid
    base = sid * ROWS_PER_TILE
    r0 = wid * WR

    pltpu.sync_copy(z8_hbm, acc1.at[pl.ds(base, ROWS_PER_TILE)])
    pltpu.sync_copy(z8_hbm, acc2.at[pl.ds(base, ROWS_PER_TILE)])
    pltpu.sync_copy(ones_hbm, ones_v)
    plsc.subcore_barrier()

    def lin_issue(bufs, k):
        d1v, d2v, isem, _ = bufs
        r = r0 + jnp.minimum(k, WRC - 1) * SUBC
        pltpu.async_copy(d1_hbm.at[pl.ds(r, SUBC)], d1v, isem)
        pltpu.async_copy(d2_hbm.at[pl.ds(r, SUBC)], d2v, isem)

    def lin_wait(bufs):
        d1v, d2v, isem, _ = bufs
        pltpu.make_async_copy(d1_hbm.at[pl.ds(r0, SUBC)], d1v, isem).wait()
        pltpu.make_async_copy(d2_hbm.at[pl.ds(r0, SUBC)], d2v, isem).wait()

    def scat_issue(bufs):
        d1v, d2v, _, ssem = bufs
        for j in range(SUBC):
            pltpu.async_copy(ones_v, acc1.at[d1v.at[j]], ssem, add=True)
            pltpu.async_copy(ones_v, acc2.at[d2v.at[j]], ssem, add=True)

    def scat_wait(bufs):
        d1v, d2v, _, ssem = bufs
        for j in range(SUBC):
            pltpu.make_async_copy(ones_v, acc1.at[d1v.at[j]], ssem).wait()
            pltpu.make_async_copy(ones_v, acc2.at[d2v.at[j]], ssem).wait()

    A = (d1a, d2a, ia, sa)
    B = (d1b, d2b, ib, sb)
    lin_issue(A, 0)
    lin_issue(B, 1)

    def body(i, carry):
        a = 2 * i
        lin_wait(A)
        scat_issue(A)
        lin_wait(B)
        scat_issue(B)
        scat_wait(A)
        lin_issue(A, a + 2)
        scat_wait(B)
        lin_issue(B, a + 3)
        return carry

    lax.fori_loop(0, NPAIRC, body, 0)
    lin_wait(A)
    lin_wait(B)

    grp = wid // 8
    grow = (wid % 8) * EPW

    def rd_issue(buf, rsem, k):
        kk = jnp.minimum(k, NRT - 1)
        pltpu.async_copy(
            tem4_hbm.at[pl.ds(grow + kk * RT, RT), pl.ds(grp * D, D)],
            buf, rsem)

    def rd_wait(buf, rsem):
        pltpu.make_async_copy(
            tem4_hbm.at[pl.ds(grow, RT), pl.ds(grp * D, D)], buf,
            rsem).wait()

    def wr_issue(buf, wsem, k):
        pltpu.async_copy(
            buf, tem_hbm.at[pl.ds(grp * (E_PAD // 4) + grow + k * RT, RT)],
            wsem)

    def wr_wait(buf, wsem):
        pltpu.make_async_copy(
            buf, tem_hbm.at[pl.ds(grp * (E_PAD // 4) + grow, RT)],
            wsem).wait()

    rd_issue(ra, rsa, 0)
    rd_issue(rb, rsb, 1)

    def dbody(i, carry):
        k = 2 * i
        rd_wait(ra, rsa)
        wr_issue(ra, wsa, k)
        rd_wait(rb, rsb)
        wr_issue(rb, wsb, k + 1)
        wr_wait(ra, wsa)
        rd_issue(ra, rsa, k + 2)
        wr_wait(rb, wsb)
        rd_issue(rb, rsb, k + 3)
        return carry

    lax.fori_loop(0, NPRT, dbody, 0)
    rd_wait(ra, rsa)
    rd_wait(rb, rsb)
    plsc.subcore_barrier()
    pltpu.sync_copy(acc1.at[pl.ds(base, ROWS_PER_TILE)],
                    c1_hbm.at[cid, pl.ds(base, ROWS_PER_TILE)])
    pltpu.sync_copy(acc2.at[pl.ds(base, ROWS_PER_TILE)],
                    c2_hbm.at[cid, pl.ds(base, ROWS_PER_TILE)])


def _counts(d1_r, d2_r, tem4, ones8, z8):
    f = pl.kernel(
        _counts_body,
        out_type=[jax.ShapeDtypeStruct((NC, ACC_ROWS, 8), jnp.float32),
                  jax.ShapeDtypeStruct((NC, ACC_ROWS, 8), jnp.float32),
                  jax.ShapeDtypeStruct((E_PAD, D), jnp.float32)],
        mesh=_mesh,
        scratch_types=[
            pltpu.VMEM_SHARED((ACC_ROWS, 8), jnp.float32),
            pltpu.VMEM_SHARED((ACC_ROWS, 8), jnp.float32),
            pltpu.VMEM((SUBC, 128), jnp.int32),
            pltpu.VMEM((SUBC, 128), jnp.int32),
            pltpu.VMEM((SUBC, 128), jnp.int32),
            pltpu.VMEM((SUBC, 128), jnp.int32),
            pltpu.VMEM((128, 8), jnp.float32),
            pltpu.VMEM((RT, D), jnp.float32),
            pltpu.VMEM((RT, D), jnp.float32),
            pltpu.SemaphoreType.DMA,
            pltpu.SemaphoreType.DMA,
            pltpu.SemaphoreType.DMA,
            pltpu.SemaphoreType.DMA,
            pltpu.SemaphoreType.DMA,
            pltpu.SemaphoreType.DMA,
            pltpu.SemaphoreType.DMA,
            pltpu.SemaphoreType.DMA,
        ],
        compiler_params=_sc_params,
    )
    return f(d1_r, d2_r, tem4, ones8, z8)


def _edge_pass_body(hw_hbm, tem_hbm, tfw_hbm, src_hbm, eidx_hbm, dst_hbm,
                    z32_hbm, p_hbm, acc,
                    sa, ea, da, tfa, ha, ta, sb, eb, db, tfb, hb, tb,
                    isa, isb, fsa, fsb, gsa, gsb, ssa, ssb):
    cid = lax.axis_index("c")
    sid = lax.axis_index("s")
    wid = sid * NC + cid
    base = sid * ROWS_PER_TILE
    r0 = wid * WR

    pltpu.sync_copy(z32_hbm, acc.at[pl.ds(base, ROWS_PER_TILE)])
    plsc.subcore_barrier()

    def se_issue(bufs, k):
        s, e, d, tf, h, t, isem, fsem, gsem, ssem = bufs
        r = r0 + jnp.minimum(k, WR - 1)
        pltpu.async_copy(src_hbm.at[pl.ds(r, 1)], s, isem)
        pltpu.async_copy(eidx_hbm.at[pl.ds(r, 1)], e, isem)

    def se_wait(bufs):
        s, e, d, tf, h, t, isem, fsem, gsem, ssem = bufs
        pltpu.make_async_copy(src_hbm.at[pl.ds(r0, 1)], s, isem).wait()
        pltpu.make_async_copy(eidx_hbm.at[pl.ds(r0, 1)], e, isem).wait()

    grp = wid // 8
    grow = (wid % 8) * EPW

    def dtf_issue(bufs, k):
        s, e, d, tf, h, t, isem, fsem, gsem, ssem = bufs
        kk = jnp.minimum(k, WR - 1)
        pltpu.async_copy(dst_hbm.at[pl.ds(r0 + kk, 1)], d, fsem)
        pltpu.async_copy(
            tfw_hbm.at[pl.ds(grow + kk * 128, CB), pl.ds(grp * D, D)],
            tf, fsem)

    def dtf_wait(bufs):
        s, e, d, tf, h, t, isem, fsem, gsem, ssem = bufs
        pltpu.make_async_copy(dst_hbm.at[pl.ds(r0, 1)], d, fsem).wait()
        pltpu.make_async_copy(
            tfw_hbm.at[pl.ds(grow, CB), pl.ds(grp * D, D)], tf, fsem).wait()

    def gat_issue(bufs):
        s, e, d, tf, h, t, isem, fsem, gsem, ssem = bufs
        pltpu.async_copy(hw_hbm.at[s.at[0]], h, gsem)
        pltpu.async_copy(tem_hbm.at[e.at[0]], t, gsem)

    def gat_wait(bufs):
        s, e, d, tf, h, t, isem, fsem, gsem, ssem = bufs
        pltpu.make_async_copy(hw_hbm.at[s.at[0]], h, gsem).wait()
        pltpu.make_async_copy(tem_hbm.at[e.at[0]], t, gsem).wait()

    def scat_issue(bufs):
        s, e, d, tf, h, t, isem, fsem, gsem, ssem = bufs
        pltpu.async_copy(tf, acc.at[d.at[0]], ssem, add=True)

    def scat_wait(bufs):
        s, e, d, tf, h, t, isem, fsem, gsem, ssem = bufs
        pltpu.make_async_copy(tf, acc.at[d.at[0]], ssem).wait()

    def compute(bufs):
        s, e, d, tf, h, t, isem, fsem, gsem, ssem = bufs

        @plsc.parallel_loop(0, CB, 1, unroll=4)
        def _(r):
            for half in (0, 16):
                v = (h[r, pl.ds(half, 16)] + t[r, pl.ds(half, 16)]
                     + tf[r, pl.ds(half, 16)])
                tf[r, pl.ds(half, 16)] = jnp.maximum(v, 0.0)

    A = (sa, ea, da, tfa, ha, ta, isa, fsa, gsa, ssa)
    B = (sb, eb, db, tfb, hb, tb, isb, fsb, gsb, ssb)

    se_issue(A, 0)
    dtf_issue(A, 0)
    se_issue(B, 1)
    se_wait(A)
    gat_issue(A)

    def body(i, carry):
        a = 2 * i

        se_wait(B)
        dtf_issue(B, a + 1)
        gat_issue(B)

        gat_wait(A)
        dtf_wait(A)
        compute(A)
        scat_issue(A)
        se_issue(A, a + 2)

        gat_wait(B)
        dtf_wait(B)
        compute(B)
        scat_issue(B)
        se_issue(B, a + 3)

        scat_wait(A)
        dtf_issue(A, a + 2)
        se_wait(A)
        gat_issue(A)

        scat_wait(B)
        return carry

    lax.fori_loop(0, NPAIR, body, 0)

    gat_wait(A)
    dtf_wait(A)
    se_wait(B)
    plsc.subcore_barrier()
    pltpu.sync_copy(acc.at[pl.ds(base, ROWS_PER_TILE)],
                    p_hbm.at[cid, pl.ds(base, ROWS_PER_TILE)])


def _edge_pass(hw, tem, tfw, src_r, eidx_r, dst_r, z32):
    f = pl.kernel(
        _edge_pass_body,
        out_type=jax.ShapeDtypeStruct((NC, ACC_ROWS, D), jnp.float32),
        mesh=_mesh,
        scratch_types=[
            pltpu.VMEM_SHARED((ACC_ROWS, D), jnp.float32),
            pltpu.VMEM((1, 128), jnp.int32),
            pltpu.VMEM((1, 128), jnp.int32),
            pltpu.VMEM((1, 128), jnp.int32),
            pltpu.VMEM((CB, D), jnp.float32),
            pltpu.VMEM((CB, D), jnp.float32),
            pltpu.VMEM((CB, D), jnp.float32),
            pltpu.VMEM((1, 128), jnp.int32),
            pltpu.VMEM((1, 128), jnp.int32),
            pltpu.VMEM((1, 128), jnp.int32),
            pltpu.VMEM((CB, D), jnp.float32),
            pltpu.VMEM((CB, D), jnp.float32),
            pltpu.VMEM((CB, D), jnp.float32),
            pltpu.SemaphoreType.DMA,
            pltpu.SemaphoreType.DMA,
            pltpu.SemaphoreType.DMA,
            pltpu.SemaphoreType.DMA,
            pltpu.SemaphoreType.DMA,
            pltpu.SemaphoreType.DMA,
            pltpu.SemaphoreType.DMA,
            pltpu.SemaphoreType.DMA,
        ],
        compiler_params=_sc_params,
    )
    return f(hw, tem, tfw, src_r, eidx_r, dst_r, z32)


def kernel(node_x, node_timestamp, edge_index, edge_rel_times, edge_idx,
           edge_mask, memory, edge_feature, w_t, b_t, W_e, b_e, W_msg, b_msg,
           W_lin, W1, b1, W2, b2, W3, b3):
    wmh = W_msg[0:D]
    wmt = W_msg[D:2 * D]
    wme = W_msg[2 * D:]
    wem = W_e @ wme
    cvec = (b_e @ wme + b_msg)[None, :]
    w_col = w_t.T
    b_col = b_t[:, None]

    src = edge_index[0]
    dst = edge_index[1]
    pe = E_PAD - E
    rel_r = jnp.pad(edge_rel_times, (0, pe)).reshape(ER, 128)
    mask_r = jnp.pad(edge_mask, (0, pe)).reshape(ER, 128)
    dstp_r = jnp.pad(dst, (0, pe), constant_values=N).reshape(ER, 128)
    src_r = jnp.pad(src, (0, pe)).reshape(ER, 128)
    eidx_r = jnp.pad(edge_idx, (0, pe)).reshape(ER, 128)

    z32 = jnp.zeros((ROWS_PER_TILE, D), jnp.float32)
    z8 = jnp.zeros((ROWS_PER_TILE, 8), jnp.float32)
    ones8 = jnp.zeros((128, 8), jnp.float32).at[:, 0].set(1.0)

    tfw = _tfw_prep(rel_r, w_col, b_col, wmt.T, cvec)
    d1_r, d2_r = _dmask(mask_r, dstp_r)
    efT = jnp.pad(edge_feature, ((0, pe), (0, 0))).T
    tem4 = _tem_prep(efT, wem.T)

    c1p, c2p, tem = _counts(d1_r, d2_r, tem4, ones8, z8)
    h0, hw0 = _node_prep(memory, node_timestamp.reshape(N // BN, 1, BN),
                         w_col, b_col, wmh)

    p1 = _edge_pass(hw0, tem, tfw, src_r, eidx_r, d1_r, z32)
    h1, hw1 = _update(h0, p1[0, :N], p1[1, :N], c1p[0, :N], c1p[1, :N],
                      W_lin, wmh)

    p2 = _edge_pass(hw1, tem, tfw, src_r, eidx_r, d2_r, z32)
    return _final(h1, p2[0, :N], p2[1, :N], c2p[0, :N], c2p[1, :N], W_lin,
                  W1, b1[None, :], W2, b2[None, :], W3, b3[None, :])


# speedup vs baseline: 1.0544x; 1.0070x over previous
import functools

import jax
import jax.numpy as jnp
from jax import lax
from jax.experimental import pallas as pl
from jax.experimental.pallas import tpu as pltpu
from jax.experimental.pallas import tpu_sc as plsc

N = 50000
E = 800000
D = 32
DE = 16

NC = 2
NS = 16
NW = NC * NS

CB = 128
EPW = 25600
E_PAD = NW * EPW
ER = E_PAD // 128
WR = EPW // 128
NPAIR = WR // 2

CBC = 1280
SUBC = CBC // 128
WRC = EPW // CBC
NPAIRC = WRC // 2

ROWS_PER_TILE = 3128
ACC_ROWS = NS * ROWS_PER_TILE

BE = 4096
BN = 2000

_mesh = plsc.VectorSubcoreMesh(core_axis_name="c", subcore_axis_name="s")
_sc_params = pltpu.CompilerParams(use_tc_tiling_on_sc=False)

_INV_2PI = 0.15915494309189535
_COS_C = (1.0, -19.739208221435547, 64.93938446044922, -85.45662689208984,
          60.24174118041992, -26.402328491210938, 7.793178081512451,
          -1.4450093507766724)


def _fast_cos(x):
    r = x * _INV_2PI
    r = r - jnp.round(r)
    u = r * r
    p = jnp.full_like(u, _COS_C[-1])
    for c in _COS_C[-2::-1]:
        p = p * u + c
    return p


QROWS = E_PAD // 4 // 128


def _tfw_body(r0_ref, r1_ref, r2_ref, r3_ref, w_ref, b_ref, wmtT_ref,
              cvec_ref, out_ref):
    parts = []
    for rel_ref in (r0_ref, r1_ref, r2_ref, r3_ref):
        rel = rel_ref[...].reshape(1, BE)
        tf = _fast_cos(w_ref[...] * rel + b_ref[...])
        t = jnp.dot(wmtT_ref[...], tf, preferred_element_type=jnp.float32)
        parts.append(t.T + cvec_ref[...])
    out_ref[...] = jnp.concatenate(parts, axis=1)


def _tfw_prep(rel_r, w_col, b_col, wmtT, cvec):
    g = E_PAD // 4 // BE
    full = lambda a: pl.BlockSpec(a.shape, lambda i: (0,) * a.ndim)
    rb = BE // 128
    rspecs = [pl.BlockSpec((rb, 128), functools.partial(
        lambda a, i: (i + a * (QROWS // rb), 0), a)) for a in range(4)]
    return pl.pallas_call(
        _tfw_body,
        grid=(g,),
        in_specs=rspecs + [full(w_col), full(b_col), full(wmtT), full(cvec)],
        out_specs=pl.BlockSpec((BE, 128), lambda i: (i, 0)),
        out_shape=jax.ShapeDtypeStruct((E_PAD // 4, 128), jnp.float32),
    )(rel_r, rel_r, rel_r, rel_r, w_col, b_col, wmtT, cvec)


def _dmask_body(mask_ref, dst_ref, d1_ref, d2_ref):
    m = mask_ref[...]
    d = dst_ref[...]
    d1_ref[...] = jnp.where(m != 0, d, N)
    d2_ref[...] = jnp.where(m != 2, d, N)


def _dmask(mask_r, dst_r):
    g = ER // 128
    return pl.pallas_call(
        _dmask_body,
        grid=(g,),
        in_specs=[pl.BlockSpec((128, 128), lambda i: (i, 0)),
                  pl.BlockSpec((128, 128), lambda i: (i, 0))],
        out_specs=[pl.BlockSpec((128, 128), lambda i: (i, 0)),
                   pl.BlockSpec((128, 128), lambda i: (i, 0))],
        out_shape=[jax.ShapeDtypeStruct((ER, 128), jnp.int32),
                   jax.ShapeDtypeStruct((ER, 128), jnp.int32)],
    )(mask_r, dst_r)


def _tem_body(e0_ref, e1_ref, e2_ref, e3_ref, wemT_ref, out_ref):
    parts = []
    for ef_ref in (e0_ref, e1_ref, e2_ref, e3_ref):
        t = jnp.dot(wemT_ref[...], ef_ref[...],
                    preferred_element_type=jnp.float32)
        parts.append(t.T)
    out_ref[...] = jnp.concatenate(parts, axis=1)


def _tem_prep(efT, wemT):
    g = E_PAD // 4 // BE
    full = lambda a: pl.BlockSpec(a.shape, lambda i: (0,) * a.ndim)
    especs = [pl.BlockSpec((DE, BE), functools.partial(
        lambda a, i: (0, i + a * g), a)) for a in range(4)]
    return pl.pallas_call(
        _tem_body,
        grid=(g,),
        in_specs=especs + [full(wemT)],
        out_specs=pl.BlockSpec((BE, 128), lambda i: (i, 0)),
        out_shape=jax.ShapeDtypeStruct((E_PAD // 4, 128), jnp.float32),
    )(efT, efT, efT, efT, wemT)


def _node_prep_body(mem_ref, ts_ref, w_ref, b_ref, wmh_ref, h0_ref, hw_ref):
    ts = ts_ref[...].reshape(1, BN)
    tf = _fast_cos(w_ref[...] * ts + b_ref[...])
    h0 = mem_ref[...] + tf.T
    h0_ref[...] = h0
    hw_ref[...] = jnp.dot(h0, wmh_ref[...], preferred_element_type=jnp.float32)


def _node_prep(memory, ts_row, w_col, b_col, wmh):
    g = N // BN
    full = lambda a: pl.BlockSpec(a.shape, lambda i: (0,) * a.ndim)
    return pl.pallas_call(
        _node_prep_body,
        grid=(g,),
        in_specs=[pl.BlockSpec((BN, D), lambda i: (i, 0)),
                  pl.BlockSpec((1, 1, BN), lambda i: (i, 0, 0)),
                  full(w_col), full(b_col), full(wmh)],
        out_specs=[pl.BlockSpec((BN, D), lambda i: (i, 0)),
                   pl.BlockSpec((BN, D), lambda i: (i, 0))],
        out_shape=[jax.ShapeDtypeStruct((N, D), jnp.float32),
                   jax.ShapeDtypeStruct((N, D), jnp.float32)],
    )(memory, ts_row, w_col, b_col, wmh)


def _update_body(h_ref, p0_ref, p1_ref, c0_ref, c1_ref, wlin_ref, wmh_ref,
                 h_out, hw_out):
    cnt = c0_ref[...][:, 0:1] + c1_ref[...][:, 0:1]
    agg = (p0_ref[...] + p1_ref[...]) / (cnt + 1.0)
    h = jax.nn.relu(jnp.dot(h_ref[...] + agg, wlin_ref[...],
                            preferred_element_type=jnp.float32))
    h_out[...] = h
    hw_out[...] = jnp.dot(h, wmh_ref[...], preferred_element_type=jnp.float32)


def _update(h, p0, p1, c0, c1, wlin, wmh):
    g = N // BN
    full = lambda a: pl.BlockSpec(a.shape, lambda i: (0,) * a.ndim)
    return pl.pallas_call(
        _update_body,
        grid=(g,),
        in_specs=[pl.BlockSpec((BN, D), lambda i: (i, 0)),
                  pl.BlockSpec((BN, D), lambda i: (i, 0)),
                  pl.BlockSpec((BN, D), lambda i: (i, 0)),
                  pl.BlockSpec((BN, 8), lambda i: (i, 0)),
                  pl.BlockSpec((BN, 8), lambda i: (i, 0)),
                  full(wlin), full(wmh)],
        out_specs=[pl.BlockSpec((BN, D), lambda i: (i, 0)),
                   pl.BlockSpec((BN, D), lambda i: (i, 0))],
        out_shape=[jax.ShapeDtypeStruct((N, D), jnp.float32),
                   jax.ShapeDtypeStruct((N, D), jnp.float32)],
    )(h, p0, p1, c0, c1, wlin, wmh)


def _final_body(h_ref, p0_ref, p1_ref, c0_ref, c1_ref, wlin_ref, w1_ref,
                b1_ref, w2_ref, b2_ref, w3_ref, b3_ref, out_ref):
    cnt = c0_ref[...][:, 0:1] + c1_ref[...][:, 0:1]
    agg = (p0_ref[...] + p1_ref[...]) / (cnt + 1.0)
    h = jax.nn.relu(jnp.dot(h_ref[...] + agg, wlin_ref[...],
                            preferred_element_type=jnp.float32))
    x = jax.nn.relu(jnp.dot(h, w1_ref[...],
                            preferred_element_type=jnp.float32) + b1_ref[...])
    x = jax.nn.relu(jnp.dot(x, w2_ref[...],
                            preferred_element_type=jnp.float32) + b2_ref[...])
    out_ref[...] = jnp.dot(x, w3_ref[...],
                           preferred_element_type=jnp.float32) + b3_ref[...]


def _final(h, p0, p1, c0, c1, wlin, w1, b1, w2, b2, w3, b3):
    g = N // BN
    full = lambda a: pl.BlockSpec(a.shape, lambda i: (0,) * a.ndim)
    return pl.pallas_call(
        _final_body,
        grid=(g,),
        in_specs=[pl.BlockSpec((BN, D), lambda i: (i, 0)),
                  pl.BlockSpec((BN, D), lambda i: (i, 0)),
                  pl.BlockSpec((BN, D), lambda i: (i, 0)),
                  pl.BlockSpec((BN, 8), lambda i: (i, 0)),
                  pl.BlockSpec((BN, 8), lambda i: (i, 0)),
                  full(wlin), full(w1), full(b1), full(w2), full(b2),
                  full(w3), full(b3)],
        out_specs=pl.BlockSpec((BN, 2), lambda i: (i, 0)),
        out_shape=jax.ShapeDtypeStruct((N, 2), jnp.float32),
    )(h, p0, p1, c0, c1, wlin, w1, b1, w2, b2, w3, b3)


RT = 512
NRT = EPW // RT
NPRT = NRT // 2


def _counts_body(d1_hbm, d2_hbm, tem4_hbm, ones_hbm, z8_hbm,
                 c1_hbm, c2_hbm, tem_hbm,
                 acc1, acc2, d1a, d2a, d1b, d2b, ones_v, ra, rb,
                 ia, ib, sa, sb, rsa, rsb, wsa, wsb):
    cid = lax.axis_index("c")
    sid = lax.axis_index("s")
    wid = sid * NC + cid
    base = sid * ROWS_PER_TILE
    r0 = wid * WR

    pltpu.sync_copy(z8_hbm, acc1.at[pl.ds(base, ROWS_PER_TILE)])
    pltpu.sync_copy(z8_hbm, acc2.at[pl.ds(base, ROWS_PER_TILE)])
    pltpu.sync_copy(ones_hbm, ones_v)
    plsc.subcore_barrier()

    def lin_issue(bufs, k):
        d1v, d2v, isem, _ = bufs
        r = r0 + jnp.minimum(k, WRC - 1) * SUBC
        pltpu.async_copy(d1_hbm.at[pl.ds(r, SUBC)], d1v, isem)
        pltpu.async_copy(d2_hbm.at[pl.ds(r, SUBC)], d2v, isem)

    def lin_wait(bufs):
        d1v, d2v, isem, _ = bufs
        pltpu.make_async_copy(d1_hbm.at[pl.ds(r0, SUBC)], d1v, isem).wait()
        pltpu.make_async_copy(d2_hbm.at[pl.ds(r0, SUBC)], d2v, isem).wait()

    def scat_issue(bufs):
        d1v, d2v, _, ssem = bufs
        for j in range(SUBC):
            pltpu.async_copy(ones_v, acc1.at[d1v.at[j]], ssem, add=True)
            pltpu.async_copy(ones_v, acc2.at[d2v.at[j]], ssem, add=True)

    def scat_wait(bufs):
        d1v, d2v, _, ssem = bufs
        for j in range(SUBC):
            pltpu.make_async_copy(ones_v, acc1.at[d1v.at[j]], ssem).wait()
            pltpu.make_async_copy(ones_v, acc2.at[d2v.at[j]], ssem).wait()

    A = (d1a, d2a, ia, sa)
    B = (d1b, d2b, ib, sb)
    lin_issue(A, 0)
    lin_issue(B, 1)

    def body(i, carry):
        a = 2 * i
        lin_wait(A)
        scat_issue(A)
        lin_wait(B)
        scat_issue(B)
        scat_wait(A)
        lin_issue(A, a + 2)
        scat_wait(B)
        lin_issue(B, a + 3)
        return carry

    lax.fori_loop(0, NPAIRC, body, 0)
    lin_wait(A)
    lin_wait(B)

    grp = wid // 8
    grow = (wid % 8) * EPW

    def rd_issue(buf, rsem, k):
        kk = jnp.minimum(k, NRT - 1)
        pltpu.async_copy(
            tem4_hbm.at[pl.ds(grow + kk * RT, RT), pl.ds(grp * D, D)],
            buf, rsem)

    def rd_wait(buf, rsem):
        pltpu.make_async_copy(
            tem4_hbm.at[pl.ds(grow, RT), pl.ds(grp * D, D)], buf,
            rsem).wait()

    def wr_issue(buf, wsem, k):
        pltpu.async_copy(
            buf, tem_hbm.at[pl.ds(grp * (E_PAD // 4) + grow + k * RT, RT)],
            wsem)

    def wr_wait(buf, wsem):
        pltpu.make_async_copy(
            buf, tem_hbm.at[pl.ds(grp * (E_PAD // 4) + grow, RT)],
            wsem).wait()

    rd_issue(ra, rsa, 0)
    rd_issue(rb, rsb, 1)

    def dbody(i, carry):
        k = 2 * i
        rd_wait(ra, rsa)
        wr_issue(ra, wsa, k)
        rd_wait(rb, rsb)
        wr_issue(rb, wsb, k + 1)
        wr_wait(ra, wsa)
        rd_issue(ra, rsa, k + 2)
        wr_wait(rb, wsb)
        rd_issue(rb, rsb, k + 3)
        return carry

    lax.fori_loop(0, NPRT, dbody, 0)
    rd_wait(ra, rsa)
    rd_wait(rb, rsb)
    plsc.subcore_barrier()
    pltpu.sync_copy(acc1.at[pl.ds(base, ROWS_PER_TILE)],
                    c1_hbm.at[cid, pl.ds(base, ROWS_PER_TILE)])
    pltpu.sync_copy(acc2.at[pl.ds(base, ROWS_PER_TILE)],
                    c2_hbm.at[cid, pl.ds(base, ROWS_PER_TILE)])


def _counts(d1_r, d2_r, tem4, ones8, z8):
    f = pl.kernel(
        _counts_body,
        out_type=[jax.ShapeDtypeStruct((NC, ACC_ROWS, 8), jnp.float32),
                  jax.ShapeDtypeStruct((NC, ACC_ROWS, 8), jnp.float32),
                  jax.ShapeDtypeStruct((E_PAD, D), jnp.float32)],
        mesh=_mesh,
        scratch_types=[
            pltpu.VMEM_SHARED((ACC_ROWS, 8), jnp.float32),
            pltpu.VMEM_SHARED((ACC_ROWS, 8), jnp.float32),
            pltpu.VMEM((SUBC, 128), jnp.int32),
            pltpu.VMEM((SUBC, 128), jnp.int32),
            pltpu.VMEM((SUBC, 128), jnp.int32),
            pltpu.VMEM((SUBC, 128), jnp.int32),
            pltpu.VMEM((128, 8), jnp.float32),
            pltpu.VMEM((RT, D), jnp.float32),
            pltpu.VMEM((RT, D), jnp.float32),
            pltpu.SemaphoreType.DMA,
            pltpu.SemaphoreType.DMA,
            pltpu.SemaphoreType.DMA,
            pltpu.SemaphoreType.DMA,
            pltpu.SemaphoreType.DMA,
            pltpu.SemaphoreType.DMA,
            pltpu.SemaphoreType.DMA,
            pltpu.SemaphoreType.DMA,
        ],
        compiler_params=_sc_params,
    )
    return f(d1_r, d2_r, tem4, ones8, z8)


def _edge_pass_body(hw_hbm, tem_hbm, tfw_hbm, src_hbm, eidx_hbm, dst_hbm,
                    z32_hbm, p_hbm, acc,
                    sa, ea, da, tfa, ha, ta, sb, eb, db, tfb, hb, tb,
                    isa, isb, fsa, fsb, gsa, gsb, ssa, ssb):
    cid = lax.axis_index("c")
    sid = lax.axis_index("s")
    wid = sid * NC + cid
    base = sid * ROWS_PER_TILE
    r0 = wid * WR

    pltpu.sync_copy(z32_hbm, acc.at[pl.ds(base, ROWS_PER_TILE)])
    plsc.subcore_barrier()

    def se_issue(bufs, k):
        s, e, d, tf, h, t, isem, fsem, gsem, ssem = bufs
        r = r0 + jnp.minimum(k, WR - 1)
        pltpu.async_copy(src_hbm.at[pl.ds(r, 1)], s, isem)
        pltpu.async_copy(eidx_hbm.at[pl.ds(r, 1)], e, isem)

    def se_wait(bufs):
        s, e, d, tf, h, t, isem, fsem, gsem, ssem = bufs
        pltpu.make_async_copy(src_hbm.at[pl.ds(r0, 1)], s, isem).wait()
        pltpu.make_async_copy(eidx_hbm.at[pl.ds(r0, 1)], e, isem).wait()

    grp = wid // 8
    grow = (wid % 8) * EPW

    def dtf_issue(bufs, k):
        s, e, d, tf, h, t, isem, fsem, gsem, ssem = bufs
        kk = jnp.minimum(k, WR - 1)
        pltpu.async_copy(dst_hbm.at[pl.ds(r0 + kk, 1)], d, fsem)
        pltpu.async_copy(
            tfw_hbm.at[pl.ds(grow + kk * 128, CB), pl.ds(grp * D, D)],
            tf, fsem)

    def dtf_wait(bufs):
        s, e, d, tf, h, t, isem, fsem, gsem, ssem = bufs
        pltpu.make_async_copy(dst_hbm.at[pl.ds(r0, 1)], d, fsem).wait()
        pltpu.make_async_copy(
            tfw_hbm.at[pl.ds(grow, CB), pl.ds(grp * D, D)], tf, fsem).wait()

    def gat_issue(bufs):
        s, e, d, tf, h, t, isem, fsem, gsem, ssem = bufs
        pltpu.async_copy(hw_hbm.at[s.at[0]], h, gsem)
        pltpu.async_copy(tem_hbm.at[e.at[0]], t, gsem)

    def gat_wait(bufs):
        s, e, d, tf, h, t, isem, fsem, gsem, ssem = bufs
        pltpu.make_async_copy(hw_hbm.at[s.at[0]], h, gsem).wait()
        pltpu.make_async_copy(tem_hbm.at[e.at[0]], t, gsem).wait()

    def scat_issue(bufs):
        s, e, d, tf, h, t, isem, fsem, gsem, ssem = bufs
        pltpu.async_copy(tf, acc.at[d.at[0]], ssem, add=True)

    def scat_wait(bufs):
        s, e, d, tf, h, t, isem, fsem, gsem, ssem = bufs
        pltpu.make_async_copy(tf, acc.at[d.at[0]], ssem).wait()

    def compute(bufs):
        s, e, d, tf, h, t, isem, fsem, gsem, ssem = bufs

        @plsc.parallel_loop(0, CB, 1, unroll=4)
        def _(r):
            for half in (0, 16):
                v = (h[r, pl.ds(half, 16)] + t[r, pl.ds(half, 16)]
                     + tf[r, pl.ds(half, 16)])
                tf[r, pl.ds(half, 16)] = jnp.maximum(v, 0.0)

    A = (sa, ea, da, tfa, ha, ta, isa, fsa, gsa, ssa)
    B = (sb, eb, db, tfb, hb, tb, isb, fsb, gsb, ssb)

    se_issue(A, 0)
    dtf_issue(A, 0)
    se_issue(B, 1)
    se_wait(A)
    gat_issue(A)

    def body(i, carry):
        a = 2 * i

        se_wait(B)
        dtf_issue(B, a + 1)
        gat_issue(B)

        gat_wait(A)
        dtf_wait(A)
        compute(A)
        scat_issue(A)
        se_issue(A, a + 2)

        gat_wait(B)
        dtf_wait(B)
        compute(B)
        scat_issue(B)
        se_issue(B, a + 3)

        scat_wait(A)
        dtf_issue(A, a + 2)
        se_wait(A)
        gat_issue(A)

        scat_wait(B)
        return carry

    lax.fori_loop(0, NPAIR, body, 0)

    gat_wait(A)
    dtf_wait(A)
    se_wait(B)
    plsc.subcore_barrier()
    pltpu.sync_copy(acc.at[pl.ds(base, ROWS_PER_TILE)],
                    p_hbm.at[cid, pl.ds(base, ROWS_PER_TILE)])


def _edge_pass(hw, tem, tfw, src_r, eidx_r, dst_r, z32):
    f = pl.kernel(
        _edge_pass_body,
        out_type=jax.ShapeDtypeStruct((NC, ACC_ROWS, D), jnp.float32),
        mesh=_mesh,
        scratch_types=[
            pltpu.VMEM_SHARED((ACC_ROWS, D), jnp.float32),
            pltpu.VMEM((1, 128), jnp.int32),
            pltpu.VMEM((1, 128), jnp.int32),
            pltpu.VMEM((1, 128), jnp.int32),
            pltpu.VMEM((CB, D), jnp.float32),
            pltpu.VMEM((CB, D), jnp.float32),
            pltpu.VMEM((CB, D), jnp.float32),
            pltpu.VMEM((1, 128), jnp.int32),
            pltpu.VMEM((1, 128), jnp.int32),
            pltpu.VMEM((1, 128), jnp.int32),
            pltpu.VMEM((CB, D), jnp.float32),
            pltpu.VMEM((CB, D), jnp.float32),
            pltpu.VMEM((CB, D), jnp.float32),
            pltpu.SemaphoreType.DMA,
            pltpu.SemaphoreType.DMA,
            pltpu.SemaphoreType.DMA,
            pltpu.SemaphoreType.DMA,
            pltpu.SemaphoreType.DMA,
            pltpu.SemaphoreType.DMA,
            pltpu.SemaphoreType.DMA,
            pltpu.SemaphoreType.DMA,
        ],
        compiler_params=_sc_params,
    )
    return f(hw, tem, tfw, src_r, eidx_r, dst_r, z32)


def kernel(node_x, node_timestamp, edge_index, edge_rel_times, edge_idx,
           edge_mask, memory, edge_feature, w_t, b_t, W_e, b_e, W_msg, b_msg,
           W_lin, W1, b1, W2, b2, W3, b3):
    wmh = W_msg[0:D]
    wmt = W_msg[D:2 * D]
    wme = W_msg[2 * D:]
    wem = W_e @ wme
    cvec = (b_e @ wme + b_msg)[None, :]
    w_col = w_t.T
    b_col = b_t[:, None]

    src = edge_index[0]
    dst = edge_index[1]
    pe = E_PAD - E
    rel_r = jnp.pad(edge_rel_times, (0, pe)).reshape(ER, 128)
    mask_r = jnp.pad(edge_mask, (0, pe)).reshape(ER, 128)
    dstp_r = jnp.pad(dst, (0, pe), constant_values=N).reshape(ER, 128)
    src_r = jnp.pad(src, (0, pe)).reshape(ER, 128)
    eidx_r = jnp.pad(edge_idx, (0, pe)).reshape(ER, 128)

    z32 = jnp.zeros((ROWS_PER_TILE, D), jnp.float32)
    z8 = jnp.zeros((ROWS_PER_TILE, 8), jnp.float32)
    ones8 = jnp.zeros((128, 8), jnp.float32).at[:, 0].set(1.0)

    tfw = _tfw_prep(rel_r, w_col, b_col, wmt.T, cvec)
    d1_r, d2_r = _dmask(mask_r, dstp_r)
    efT = jnp.pad(edge_feature, ((0, pe), (0, 0))).T
    tem4 = _tem_prep(efT, wem.T)

    c1p, c2p, tem = _counts(d1_r, d2_r, tem4, ones8, z8)
    h0, hw0 = _node_prep(memory, node_timestamp.reshape(N // BN, 1, BN),
                         w_col, b_col, wmh)

    p1 = _edge_pass(hw0, tem, tfw, src_r, eidx_r, d1_r, z32)
    h1, hw1 = _update(h0, p1[0, :N], p1[1, :N], c1p[0, :N], c1p[1, :N],
                      W_lin, wmh)

    p2 = _edge_pass(hw1, tem, tfw, src_r, eidx_r, d2_r, z32)
    return _final(h1, p2[0, :N], p2[1, :N], c2p[0, :N], c2p[1, :N], W_lin,
                  W1, b1[None, :], W2, b2[None, :], W3, b3[None, :])


# speedup vs baseline: 1.0557x; 1.0012x over previous
import functools

import jax
import jax.numpy as jnp
from jax import lax
from jax.experimental import pallas as pl
from jax.experimental.pallas import tpu as pltpu
from jax.experimental.pallas import tpu_sc as plsc

N = 50000
E = 800000
D = 32
DE = 16

NC = 2
NS = 16
NW = NC * NS

CB = 128
EPW = 25600
E_PAD = NW * EPW
ER = E_PAD // 128
WR = EPW // 128
NPAIR = WR // 2

CBC = 1280
SUBC = CBC // 128
WRC = EPW // CBC
NPAIRC = WRC // 2

ROWS_PER_TILE = 3128
ACC_ROWS = NS * ROWS_PER_TILE

BE = 4096
BN = 2000

_mesh = plsc.VectorSubcoreMesh(core_axis_name="c", subcore_axis_name="s")
_sc_params = pltpu.CompilerParams(use_tc_tiling_on_sc=False)

_INV_2PI = 0.15915494309189535
_COS_C = (1.0, -19.739208221435547, 64.93938446044922, -85.45662689208984,
          60.24174118041992, -26.402328491210938, 7.793178081512451,
          -1.4450093507766724)


def _fast_cos(x):
    r = x * _INV_2PI
    r = r - jnp.round(r)
    u = r * r
    p = jnp.full_like(u, _COS_C[-1])
    for c in _COS_C[-2::-1]:
        p = p * u + c
    return p


QROWS = E_PAD // 4 // 128


def _tfw_body(r0_ref, r1_ref, r2_ref, r3_ref, w_ref, b_ref, wmtT_ref,
              cvec_ref, out_ref):
    parts = []
    for rel_ref in (r0_ref, r1_ref, r2_ref, r3_ref):
        rel = rel_ref[...].reshape(1, BE)
        tf = _fast_cos(w_ref[...] * rel + b_ref[...])
        t = jnp.dot(wmtT_ref[...], tf, preferred_element_type=jnp.float32)
        parts.append(t.T + cvec_ref[...])
    out_ref[...] = jnp.concatenate(parts, axis=1)


def _tfw_prep(rel_r, w_col, b_col, wmtT, cvec):
    g = E_PAD // 4 // BE
    full = lambda a: pl.BlockSpec(a.shape, lambda i: (0,) * a.ndim)
    rb = BE // 128
    rspecs = [pl.BlockSpec((rb, 128), functools.partial(
        lambda a, i: (i + a * (QROWS // rb), 0), a)) for a in range(4)]
    return pl.pallas_call(
        _tfw_body,
        grid=(g,),
        in_specs=rspecs + [full(w_col), full(b_col), full(wmtT), full(cvec)],
        out_specs=pl.BlockSpec((BE, 128), lambda i: (i, 0)),
        out_shape=jax.ShapeDtypeStruct((E_PAD // 4, 128), jnp.float32),
    )(rel_r, rel_r, rel_r, rel_r, w_col, b_col, wmtT, cvec)


def _dmask_body(mask_ref, dst_ref, d1_ref, d2_ref):
    m = mask_ref[...]
    d = dst_ref[...]
    d1_ref[...] = jnp.where(m != 0, d, N)
    d2_ref[...] = jnp.where(m != 2, d, N)


def _dmask(mask_r, dst_r):
    g = ER // 128
    return pl.pallas_call(
        _dmask_body,
        grid=(g,),
        in_specs=[pl.BlockSpec((128, 128), lambda i: (i, 0)),
                  pl.BlockSpec((128, 128), lambda i: (i, 0))],
        out_specs=[pl.BlockSpec((128, 128), lambda i: (i, 0)),
                   pl.BlockSpec((128, 128), lambda i: (i, 0))],
        out_shape=[jax.ShapeDtypeStruct((ER, 128), jnp.int32),
                   jax.ShapeDtypeStruct((ER, 128), jnp.int32)],
    )(mask_r, dst_r)


def _tem_body(e0_ref, e1_ref, e2_ref, e3_ref, wemT_ref, out_ref):
    parts = []
    for ef_ref in (e0_ref, e1_ref, e2_ref, e3_ref):
        t = jnp.dot(wemT_ref[...], ef_ref[...],
                    preferred_element_type=jnp.float32)
        parts.append(t.T)
    out_ref[...] = jnp.concatenate(parts, axis=1)


def _tem_prep(efT, wemT):
    g = E_PAD // 4 // BE
    full = lambda a: pl.BlockSpec(a.shape, lambda i: (0,) * a.ndim)
    especs = [pl.BlockSpec((DE, BE), functools.partial(
        lambda a, i: (0, i + a * g), a)) for a in range(4)]
    return pl.pallas_call(
        _tem_body,
        grid=(g,),
        in_specs=especs + [full(wemT)],
        out_specs=pl.BlockSpec((BE, 128), lambda i: (i, 0)),
        out_shape=jax.ShapeDtypeStruct((E_PAD // 4, 128), jnp.float32),
    )(efT, efT, efT, efT, wemT)


def _node_prep_body(mem_ref, ts_ref, w_ref, b_ref, wmh_ref, h0_ref, hw_ref):
    ts = ts_ref[...].reshape(1, BN)
    tf = _fast_cos(w_ref[...] * ts + b_ref[...])
    h0 = mem_ref[...] + tf.T
    h0_ref[...] = h0
    hw_ref[...] = jnp.dot(h0, wmh_ref[...], preferred_element_type=jnp.float32)


def _node_prep(memory, ts_row, w_col, b_col, wmh):
    g = N // BN
    full = lambda a: pl.BlockSpec(a.shape, lambda i: (0,) * a.ndim)
    return pl.pallas_call(
        _node_prep_body,
        grid=(g,),
        in_specs=[pl.BlockSpec((BN, D), lambda i: (i, 0)),
                  pl.BlockSpec((1, 1, BN), lambda i: (i, 0, 0)),
                  full(w_col), full(b_col), full(wmh)],
        out_specs=[pl.BlockSpec((BN, D), lambda i: (i, 0)),
                   pl.BlockSpec((BN, D), lambda i: (i, 0))],
        out_shape=[jax.ShapeDtypeStruct((N, D), jnp.float32),
                   jax.ShapeDtypeStruct((N, D), jnp.float32)],
    )(memory, ts_row, w_col, b_col, wmh)


def _update_body(h_ref, p0_ref, p1_ref, c0_ref, c1_ref, wlin_ref, wmh_ref,
                 h_out, hw_out):
    cnt = c0_ref[...][:, 0:1] + c1_ref[...][:, 0:1]
    agg = (p0_ref[...] + p1_ref[...]) / (cnt + 1.0)
    h = jax.nn.relu(jnp.dot(h_ref[...] + agg, wlin_ref[...],
                            preferred_element_type=jnp.float32))
    h_out[...] = h
    hw_out[...] = jnp.dot(h, wmh_ref[...], preferred_element_type=jnp.float32)


def _update(h, p0, p1, c0, c1, wlin, wmh):
    g = N // BN
    full = lambda a: pl.BlockSpec(a.shape, lambda i: (0,) * a.ndim)
    return pl.pallas_call(
        _update_body,
        grid=(g,),
        in_specs=[pl.BlockSpec((BN, D), lambda i: (i, 0)),
                  pl.BlockSpec((BN, D), lambda i: (i, 0)),
                  pl.BlockSpec((BN, D), lambda i: (i, 0)),
                  pl.BlockSpec((BN, 8), lambda i: (i, 0)),
                  pl.BlockSpec((BN, 8), lambda i: (i, 0)),
                  full(wlin), full(wmh)],
        out_specs=[pl.BlockSpec((BN, D), lambda i: (i, 0)),
                   pl.BlockSpec((BN, D), lambda i: (i, 0))],
        out_shape=[jax.ShapeDtypeStruct((N, D), jnp.float32),
                   jax.ShapeDtypeStruct((N, D), jnp.float32)],
    )(h, p0, p1, c0, c1, wlin, wmh)


def _final_body(h_ref, p0_ref, p1_ref, c0_ref, c1_ref, wlin_ref, w1_ref,
                b1_ref, w2_ref, b2_ref, w3_ref, b3_ref, out_ref):
    cnt = c0_ref[...][:, 0:1] + c1_ref[...][:, 0:1]
    agg = (p0_ref[...] + p1_ref[...]) / (cnt + 1.0)
    h = jax.nn.relu(jnp.dot(h_ref[...] + agg, wlin_ref[...],
                            preferred_element_type=jnp.float32))
    x = jax.nn.relu(jnp.dot(h, w1_ref[...],
                            preferred_element_type=jnp.float32) + b1_ref[...])
    x = jax.nn.relu(jnp.dot(x, w2_ref[...],
                            preferred_element_type=jnp.float32) + b2_ref[...])
    out_ref[...] = jnp.dot(x, w3_ref[...],
                           preferred_element_type=jnp.float32) + b3_ref[...]


def _final(h, p0, p1, c0, c1, wlin, w1, b1, w2, b2, w3, b3):
    g = N // BN
    full = lambda a: pl.BlockSpec(a.shape, lambda i: (0,) * a.ndim)
    return pl.pallas_call(
        _final_body,
        grid=(g,),
        in_specs=[pl.BlockSpec((BN, D), lambda i: (i, 0)),
                  pl.BlockSpec((BN, D), lambda i: (i, 0)),
                  pl.BlockSpec((BN, D), lambda i: (i, 0)),
                  pl.BlockSpec((BN, 8), lambda i: (i, 0)),
                  pl.BlockSpec((BN, 8), lambda i: (i, 0)),
                  full(wlin), full(w1), full(b1), full(w2), full(b2),
                  full(w3), full(b3)],
        out_specs=pl.BlockSpec((BN, 2), lambda i: (i, 0)),
        out_shape=jax.ShapeDtypeStruct((N, 2), jnp.float32),
    )(h, p0, p1, c0, c1, wlin, w1, b1, w2, b2, w3, b3)


RT = 512
NRT = EPW // RT
NPRT = NRT // 2


def _counts_body(d1_hbm, d2_hbm, tem4_hbm, ones_hbm, z8_hbm,
                 c1_hbm, c2_hbm, tem_hbm,
                 acc1, acc2, d1a, d2a, d1b, d2b, ones_v, ra, rb,
                 ia, ib, sa, sb, rsa, rsb, wsa, wsb):
    cid = lax.axis_index("c")
    sid = lax.axis_index("s")
    wid = sid * NC + cid
    base = sid * ROWS_PER_TILE
    r0 = wid * WR

    pltpu.sync_copy(z8_hbm, acc1.at[pl.ds(base, ROWS_PER_TILE)])
    pltpu.sync_copy(z8_hbm, acc2.at[pl.ds(base, ROWS_PER_TILE)])
    pltpu.sync_copy(ones_hbm, ones_v)
    plsc.subcore_barrier()

    def lin_issue(bufs, k):
        d1v, d2v, isem, _ = bufs
        r = r0 + jnp.minimum(k, WRC - 1) * SUBC
        pltpu.async_copy(d1_hbm.at[pl.ds(r, SUBC)], d1v, isem)
        pltpu.async_copy(d2_hbm.at[pl.ds(r, SUBC)], d2v, isem)

    def lin_wait(bufs):
        d1v, d2v, isem, _ = bufs
        pltpu.make_async_copy(d1_hbm.at[pl.ds(r0, SUBC)], d1v, isem).wait()
        pltpu.make_async_copy(d2_hbm.at[pl.ds(r0, SUBC)], d2v, isem).wait()

    def scat_issue(bufs):
        d1v, d2v, _, ssem = bufs
        for j in range(SUBC):
            pltpu.async_copy(ones_v, acc1.at[d1v.at[j]], ssem, add=True)
            pltpu.async_copy(ones_v, acc2.at[d2v.at[j]], ssem, add=True)

    def scat_wait(bufs):
        d1v, d2v, _, ssem = bufs
        for j in range(SUBC):
            pltpu.make_async_copy(ones_v, acc1.at[d1v.at[j]], ssem).wait()
            pltpu.make_async_copy(ones_v, acc2.at[d2v.at[j]], ssem).wait()

    A = (d1a, d2a, ia, sa)
    B = (d1b, d2b, ib, sb)
    lin_issue(A, 0)
    lin_issue(B, 1)

    grp = wid // 8
    grow = (wid % 8) * EPW

    def rd_issue(buf, rsem, k):
        kk = jnp.minimum(k, NRT - 1)
        pltpu.async_copy(
            tem4_hbm.at[pl.ds(grow + kk * RT, RT), pl.ds(grp * D, D)],
            buf, rsem)

    def rd_wait(buf, rsem):
        pltpu.make_async_copy(
            tem4_hbm.at[pl.ds(grow, RT), pl.ds(grp * D, D)], buf,
            rsem).wait()

    def wr_issue(buf, wsem, k):
        pltpu.async_copy(
            buf, tem_hbm.at[pl.ds(grp * (E_PAD // 4) + grow + k * RT, RT)],
            wsem)

    def wr_wait(buf, wsem):
        pltpu.make_async_copy(
            buf, tem_hbm.at[pl.ds(grp * (E_PAD // 4) + grow, RT)],
            wsem).wait()

    rd_issue(ra, rsa, 0)
    rd_issue(rb, rsb, 1)

    def dbody(i, carry):
        k = 2 * i

        @pl.when(i < NPAIRC)
        def _():
            a = 2 * i
            lin_wait(A)
            scat_issue(A)
            lin_wait(B)
            scat_issue(B)
            scat_wait(A)
            lin_issue(A, a + 2)
            scat_wait(B)
            lin_issue(B, a + 3)

        rd_wait(ra, rsa)
        wr_issue(ra, wsa, k)
        rd_wait(rb, rsb)
        wr_issue(rb, wsb, k + 1)
        wr_wait(ra, wsa)
        rd_issue(ra, rsa, k + 2)
        wr_wait(rb, wsb)
        rd_issue(rb, rsb, k + 3)
        return carry

    lax.fori_loop(0, NPRT, dbody, 0)
    lin_wait(A)
    lin_wait(B)
    rd_wait(ra, rsa)
    rd_wait(rb, rsb)
    plsc.subcore_barrier()
    pltpu.sync_copy(acc1.at[pl.ds(base, ROWS_PER_TILE)],
                    c1_hbm.at[cid, pl.ds(base, ROWS_PER_TILE)])
    pltpu.sync_copy(acc2.at[pl.ds(base, ROWS_PER_TILE)],
                    c2_hbm.at[cid, pl.ds(base, ROWS_PER_TILE)])


def _counts(d1_r, d2_r, tem4, ones8, z8):
    f = pl.kernel(
        _counts_body,
        out_type=[jax.ShapeDtypeStruct((NC, ACC_ROWS, 8), jnp.float32),
                  jax.ShapeDtypeStruct((NC, ACC_ROWS, 8), jnp.float32),
                  jax.ShapeDtypeStruct((E_PAD, D), jnp.float32)],
        mesh=_mesh,
        scratch_types=[
            pltpu.VMEM_SHARED((ACC_ROWS, 8), jnp.float32),
            pltpu.VMEM_SHARED((ACC_ROWS, 8), jnp.float32),
            pltpu.VMEM((SUBC, 128), jnp.int32),
            pltpu.VMEM((SUBC, 128), jnp.int32),
            pltpu.VMEM((SUBC, 128), jnp.int32),
            pltpu.VMEM((SUBC, 128), jnp.int32),
            pltpu.VMEM((128, 8), jnp.float32),
            pltpu.VMEM((RT, D), jnp.float32),
            pltpu.VMEM((RT, D), jnp.float32),
            pltpu.SemaphoreType.DMA,
            pltpu.SemaphoreType.DMA,
            pltpu.SemaphoreType.DMA,
            pltpu.SemaphoreType.DMA,
            pltpu.SemaphoreType.DMA,
            pltpu.SemaphoreType.DMA,
            pltpu.SemaphoreType.DMA,
            pltpu.SemaphoreType.DMA,
        ],
        compiler_params=_sc_params,
    )
    return f(d1_r, d2_r, tem4, ones8, z8)


def _edge_pass_body(hw_hbm, tem_hbm, tfw_hbm, src_hbm, eidx_hbm, dst_hbm,
                    z32_hbm, p_hbm, acc,
                    sa, ea, da, tfa, ha, ta, sb, eb, db, tfb, hb, tb,
                    isa, isb, fsa, fsb, gsa, gsb, ssa, ssb):
    cid = lax.axis_index("c")
    sid = lax.axis_index("s")
    wid = sid * NC + cid
    base = sid * ROWS_PER_TILE
    r0 = wid * WR

    pltpu.sync_copy(z32_hbm, acc.at[pl.ds(base, ROWS_PER_TILE)])
    plsc.subcore_barrier()

    def se_issue(bufs, k):
        s, e, d, tf, h, t, isem, fsem, gsem, ssem = bufs
        r = r0 + jnp.minimum(k, WR - 1)
        pltpu.async_copy(src_hbm.at[pl.ds(r, 1)], s, isem)
        pltpu.async_copy(eidx_hbm.at[pl.ds(r, 1)], e, isem)

    def se_wait(bufs):
        s, e, d, tf, h, t, isem, fsem, gsem, ssem = bufs
        pltpu.make_async_copy(src_hbm.at[pl.ds(r0, 1)], s, isem).wait()
        pltpu.make_async_copy(eidx_hbm.at[pl.ds(r0, 1)], e, isem).wait()

    grp = wid // 8
    grow = (wid % 8) * EPW

    def dtf_issue(bufs, k):
        s, e, d, tf, h, t, isem, fsem, gsem, ssem = bufs
        kk = jnp.minimum(k, WR - 1)
        pltpu.async_copy(dst_hbm.at[pl.ds(r0 + kk, 1)], d, fsem)
        pltpu.async_copy(
            tfw_hbm.at[pl.ds(grow + kk * 128, CB), pl.ds(grp * D, D)],
            tf, fsem)

    def dtf_wait(bufs):
        s, e, d, tf, h, t, isem, fsem, gsem, ssem = bufs
        pltpu.make_async_copy(dst_hbm.at[pl.ds(r0, 1)], d, fsem).wait()
        pltpu.make_async_copy(
            tfw_hbm.at[pl.ds(grow, CB), pl.ds(grp * D, D)], tf, fsem).wait()

    def gat_issue(bufs):
        s, e, d, tf, h, t, isem, fsem, gsem, ssem = bufs
        pltpu.async_copy(hw_hbm.at[s.at[0]], h, gsem)
        pltpu.async_copy(tem_hbm.at[e.at[0]], t, gsem)

    def gat_wait(bufs):
        s, e, d, tf, h, t, isem, fsem, gsem, ssem = bufs
        pltpu.make_async_copy(hw_hbm.at[s.at[0]], h, gsem).wait()
        pltpu.make_async_copy(tem_hbm.at[e.at[0]], t, gsem).wait()

    def scat_issue(bufs):
        s, e, d, tf, h, t, isem, fsem, gsem, ssem = bufs
        pltpu.async_copy(tf, acc.at[d.at[0]], ssem, add=True)

    def scat_wait(bufs):
        s, e, d, tf, h, t, isem, fsem, gsem, ssem = bufs
        pltpu.make_async_copy(tf, acc.at[d.at[0]], ssem).wait()

    def compute(bufs):
        s, e, d, tf, h, t, isem, fsem, gsem, ssem = bufs

        @plsc.parallel_loop(0, CB, 1, unroll=4)
        def _(r):
            for half in (0, 16):
                v = (h[r, pl.ds(half, 16)] + t[r, pl.ds(half, 16)]
                     + tf[r, pl.ds(half, 16)])
                tf[r, pl.ds(half, 16)] = jnp.maximum(v, 0.0)

    A = (sa, ea, da, tfa, ha, ta, isa, fsa, gsa, ssa)
    B = (sb, eb, db, tfb, hb, tb, isb, fsb, gsb, ssb)

    se_issue(A, 0)
    dtf_issue(A, 0)
    se_issue(B, 1)
    se_wait(A)
    gat_issue(A)

    def body(i, carry):
        a = 2 * i

        se_wait(B)
        dtf_issue(B, a + 1)
        gat_issue(B)

        gat_wait(A)
        dtf_wait(A)
        compute(A)
        scat_issue(A)
        se_issue(A, a + 2)

        gat_wait(B)
        dtf_wait(B)
        compute(B)
        scat_issue(B)
        se_issue(B, a + 3)

        scat_wait(A)
        dtf_issue(A, a + 2)
        se_wait(A)
        gat_issue(A)

        scat_wait(B)
        return carry

    lax.fori_loop(0, NPAIR, body, 0)

    gat_wait(A)
    dtf_wait(A)
    se_wait(B)
    plsc.subcore_barrier()
    pltpu.sync_copy(acc.at[pl.ds(base, ROWS_PER_TILE)],
                    p_hbm.at[cid, pl.ds(base, ROWS_PER_TILE)])


def _edge_pass(hw, tem, tfw, src_r, eidx_r, dst_r, z32):
    f = pl.kernel(
        _edge_pass_body,
        out_type=jax.ShapeDtypeStruct((NC, ACC_ROWS, D), jnp.float32),
        mesh=_mesh,
        scratch_types=[
            pltpu.VMEM_SHARED((ACC_ROWS, D), jnp.float32),
            pltpu.VMEM((1, 128), jnp.int32),
            pltpu.VMEM((1, 128), jnp.int32),
            pltpu.VMEM((1, 128), jnp.int32),
            pltpu.VMEM((CB, D), jnp.float32),
            pltpu.VMEM((CB, D), jnp.float32),
            pltpu.VMEM((CB, D), jnp.float32),
            pltpu.VMEM((1, 128), jnp.int32),
            pltpu.VMEM((1, 128), jnp.int32),
            pltpu.VMEM((1, 128), jnp.int32),
            pltpu.VMEM((CB, D), jnp.float32),
            pltpu.VMEM((CB, D), jnp.float32),
            pltpu.VMEM((CB, D), jnp.float32),
            pltpu.SemaphoreType.DMA,
            pltpu.SemaphoreType.DMA,
            pltpu.SemaphoreType.DMA,
            pltpu.SemaphoreType.DMA,
            pltpu.SemaphoreType.DMA,
            pltpu.SemaphoreType.DMA,
            pltpu.SemaphoreType.DMA,
            pltpu.SemaphoreType.DMA,
        ],
        compiler_params=_sc_params,
    )
    return f(hw, tem, tfw, src_r, eidx_r, dst_r, z32)


def kernel(node_x, node_timestamp, edge_index, edge_rel_times, edge_idx,
           edge_mask, memory, edge_feature, w_t, b_t, W_e, b_e, W_msg, b_msg,
           W_lin, W1, b1, W2, b2, W3, b3):
    wmh = W_msg[0:D]
    wmt = W_msg[D:2 * D]
    wme = W_msg[2 * D:]
    wem = W_e @ wme
    cvec = (b_e @ wme + b_msg)[None, :]
    w_col = w_t.T
    b_col = b_t[:, None]

    src = edge_index[0]
    dst = edge_index[1]
    pe = E_PAD - E
    rel_r = jnp.pad(edge_rel_times, (0, pe)).reshape(ER, 128)
    mask_r = jnp.pad(edge_mask, (0, pe)).reshape(ER, 128)
    dstp_r = jnp.pad(dst, (0, pe), constant_values=N).reshape(ER, 128)
    src_r = jnp.pad(src, (0, pe)).reshape(ER, 128)
    eidx_r = jnp.pad(edge_idx, (0, pe)).reshape(ER, 128)

    z32 = jnp.zeros((ROWS_PER_TILE, D), jnp.float32)
    z8 = jnp.zeros((ROWS_PER_TILE, 8), jnp.float32)
    ones8 = jnp.zeros((128, 8), jnp.float32).at[:, 0].set(1.0)

    tfw = _tfw_prep(rel_r, w_col, b_col, wmt.T, cvec)
    d1_r, d2_r = _dmask(mask_r, dstp_r)
    efT = jnp.pad(edge_feature, ((0, pe), (0, 0))).T
    tem4 = _tem_prep(efT, wem.T)

    c1p, c2p, tem = _counts(d1_r, d2_r, tem4, ones8, z8)
    h0, hw0 = _node_prep(memory, node_timestamp.reshape(N // BN, 1, BN),
                         w_col, b_col, wmh)

    p1 = _edge_pass(hw0, tem, tfw, src_r, eidx_r, d1_r, z32)
    h1, hw1 = _update(h0, p1[0, :N], p1[1, :N], c1p[0, :N], c1p[1, :N],
                      W_lin, wmh)

    p2 = _edge_pass(hw1, tem, tfw, src_r, eidx_r, d2_r, z32)
    return _final(h1, p2[0, :N], p2[1, :N], c2p[0, :N], c2p[1, :N], W_lin,
                  W1, b1[None, :], W2, b2[None, :], W3, b3[None, :])
